# baseline (speedup 1.0000x reference)
import functools

import jax
import jax.numpy as jnp
from jax import lax
from jax.experimental import pallas as pl
from jax.experimental.pallas import tpu as pltpu
from jax.experimental.pallas import tpu_sc as plsc

_NT = ["customer", "transaction", "device", "email", "address"]
_NN = {"customer": 50000, "transaction": 100000, "device": 10000,
       "email": 10000, "address": 10000}
_H = 64
_W = 128
_E = 150000
_RELS = [
    ("makes", "customer", "transaction"),
    ("used_in", "device", "transaction"),
    ("linked_to", "email", "transaction"),
    ("located_at", "address", "transaction"),
    ("rev_makes", "transaction", "customer"),
    ("rev_used_in", "transaction", "device"),
    ("rev_linked_to", "transaction", "email"),
    ("rev_located_at", "transaction", "address"),
]

_NSUB = 16
_EPT = 9472
_EP = _NSUB * _EPT
_NBLK = _EPT // 128
_CHUNK = {"transaction": (8448, 6, 101376),
          "customer": (8448, 3, 50688),
          "device": (5120, 1, 10240),
          "email": (5120, 1, 10240),
          "address": (5120, 1, 10240)}
_ACC_ROWS = 8464
_ZROWS = 64


def _sc_layer_kernel():
  mesh = plsc.VectorSubcoreMesh(core_axis_name="c", subcore_axis_name="s")
  out_type = [jax.ShapeDtypeStruct((_CHUNK[d][2], _W), jnp.float32)
              for (_, _, d) in _RELS]

  scratch = [
      pltpu.VMEM_SHARED((_ACC_ROWS, _W), jnp.float32),
      pltpu.VMEM((_EPT,), jnp.int32),
      pltpu.VMEM((_EPT,), jnp.int32),
      pltpu.VMEM((_NBLK, 128), jnp.int32),
      pltpu.VMEM((128, _W), jnp.float32),
      pltpu.VMEM((_ZROWS, _W), jnp.float32),
      pltpu.SemaphoreType.DMA,
  ]

  def body(*refs):
    h_refs = refs[0:5]
    src_refs = refs[5:13]
    dst_refs = refs[13:21]
    out_refs = refs[21:29]
    (acc_sh, srcw, dstw, comp, rows, zb, sem) = refs[29:]

    s_idx = lax.axis_index("s")
    c_idx = lax.axis_index("c")

    zf = jnp.zeros((16,), jnp.float32)

    def fill_zb(r, _):
      for l in range(_W // 16):
        zb[r, pl.ds(l * 16, 16)] = zf
      return 0
    lax.fori_loop(0, _ZROWS, fill_zb, 0)

    h_by_nt = dict(zip(_NT, h_refs))

    for ri, (_, s_nt, d_nt) in enumerate(_RELS):
      cr, m, _ = _CHUNK[d_nt]
      h_ref = h_by_nt[s_nt]
      agg_ref = out_refs[ri]
      rpt = cr // _NSUB
      share0 = s_idx * rpt

      pltpu.sync_copy(src_refs[ri].at[pl.ds(s_idx * _EPT, _EPT)], srcw)
      pltpu.sync_copy(dst_refs[ri].at[pl.ds(s_idx * _EPT, _EPT)], dstw)

      def chunk(k, _):
        lo = (2 * k + c_idx) * cr

        nfull, rem = divmod(rpt, _ZROWS)
        for z in range(nfull):
          pltpu.sync_copy(zb, acc_sh.at[pl.ds(share0 + z * _ZROWS, _ZROWS)])
        if rem:
          pltpu.sync_copy(zb.at[pl.ds(0, rem)],
                          acc_sh.at[pl.ds(share0 + nfull * _ZROWS, rem)])
        plsc.subcore_barrier()

        def remap(r, _):
          for l in range(8):
            d = dstw[pl.ds(r * 128 + l * 16, 16)]
            inr = (d >= lo) & (d < lo + cr)
            comp[r, pl.ds(l * 16, 16)] = jnp.where(inr, d - lo, cr)
          return 0
        lax.fori_loop(0, _NBLK, remap, 0)

        def blk(b, _):
          pltpu.async_copy(h_ref.at[srcw.at[pl.ds(b * 128, 128)]],
                           rows, sem).wait()
          pltpu.sync_copy(rows, acc_sh.at[comp.at[b]], add=True)
          return 0
        lax.fori_loop(0, _NBLK, blk, 0)
        plsc.subcore_barrier()

        pltpu.sync_copy(acc_sh.at[pl.ds(share0, rpt)],
                        agg_ref.at[pl.ds(lo + share0, rpt)])
        return 0

      lax.fori_loop(0, m, chunk, 0)

  return pl.kernel(body, out_type=out_type, mesh=mesh, scratch_types=scratch)


_BR = 1000


def _proj(x, w, b):
  n, kd = x.shape
  ho = w.shape[0]
  t = n // _BR

  def body(x_ref, w_ref, b_ref, z_ref, st_ref):
    z = jnp.dot(x_ref[...], w_ref[...].T,
                preferred_element_type=jnp.float32) + b_ref[...]
    z_ref[...] = z
    st_ref[...] = jnp.stack([jnp.sum(z, 0), jnp.sum(z * z, 0)])[None]

  return pl.pallas_call(
      body,
      grid=(t,),
      in_specs=[
          pl.BlockSpec((_BR, kd), lambda i: (i, 0)),
          pl.BlockSpec((ho, kd), lambda i: (0, 0)),
          pl.BlockSpec((1, ho), lambda i: (0, 0)),
      ],
      out_specs=[
          pl.BlockSpec((_BR, ho), lambda i: (i, 0)),
          pl.BlockSpec((1, 2, ho), lambda i: (i, 0, 0)),
      ],
      out_shape=[
          jax.ShapeDtypeStruct((n, ho), jnp.float32),
          jax.ShapeDtypeStruct((t, 2, ho), jnp.float32),
      ],
  )(x, w, b.reshape(1, ho))


def _combine(h, wr_eff, bl_eff, wl_st, aggs, n):
  nrel = wl_st.shape[0]
  t = n // _BR

  def body(*refs):
    h_ref, wr_ref, bl_ref, wl_ref = refs[0:4]
    agg_refs = refs[4:4 + nrel]
    z_ref, st_ref = refs[4 + nrel:]
    z = jnp.dot(h_ref[...], wr_ref[...].T,
                preferred_element_type=jnp.float32) + bl_ref[...]
    for r in range(nrel):
      a = agg_refs[r][...]
      cnt = jnp.maximum(a[:, 64:65], 1.0)
      mean = a / cnt
      z = z + jnp.dot(mean, wl_ref[r].T, preferred_element_type=jnp.float32)
    z_ref[...] = z
    st_ref[...] = jnp.stack([jnp.sum(z, 0), jnp.sum(z * z, 0)])[None]

  in_specs = [
      pl.BlockSpec((_BR, _W), lambda i: (i, 0)),
      pl.BlockSpec((_W, _W), lambda i: (0, 0)),
      pl.BlockSpec((1, _W), lambda i: (0, 0)),
      pl.BlockSpec((nrel, _W, _W), lambda i: (0, 0, 0)),
  ]
  in_specs += [pl.BlockSpec((_BR, _W), lambda i: (i, 0))] * nrel

  return pl.pallas_call(
      body,
      grid=(t,),
      in_specs=in_specs,
      out_specs=[
          pl.BlockSpec((_BR, _W), lambda i: (i, 0)),
          pl.BlockSpec((1, 2, _W), lambda i: (i, 0, 0)),
      ],
      out_shape=[
          jax.ShapeDtypeStruct((n, _W), jnp.float32),
          jax.ShapeDtypeStruct((t, 2, _W), jnp.float32),
      ],
  )(h, wr_eff, bl_eff.reshape(1, _W), wl_st, *aggs)


def _apply_bn_relu(z, st, g, beta, n):
  ho = z.shape[1]
  t = n // _BR

  def body(z_ref, st_ref, g_ref, b_ref, h_ref):
    stf = st_ref[...]
    inv_n = 1.0 / n
    m = jnp.sum(stf[:, 0, :], 0) * inv_n
    ms2 = jnp.sum(stf[:, 1, :], 0) * inv_n
    var = ms2 - m * m
    sc = g_ref[...][0] * lax.rsqrt(var + 1e-5)
    sh = b_ref[...][0] - m * sc
    h_ref[...] = jnp.maximum(z_ref[...] * sc + sh, 0.0)

  return pl.pallas_call(
      body,
      grid=(t,),
      in_specs=[
          pl.BlockSpec((_BR, ho), lambda i: (i, 0)),
          pl.BlockSpec((t, 2, ho), lambda i: (0, 0, 0)),
          pl.BlockSpec((1, ho), lambda i: (0, 0)),
          pl.BlockSpec((1, ho), lambda i: (0, 0)),
      ],
      out_specs=pl.BlockSpec((_BR, ho), lambda i: (i, 0)),
      out_shape=jax.ShapeDtypeStruct((n, ho), jnp.float32),
  )(z, st, g.reshape(1, ho), beta.reshape(1, ho))


def _apply_bn_relu_sig(z, st, g, beta, w3p, b3p, n):
  ho = z.shape[1]
  t = n // _BR

  def body(z_ref, st_ref, g_ref, b_ref, w3_ref, b3_ref, o_ref):
    stf = st_ref[...]
    inv_n = 1.0 / n
    m = jnp.sum(stf[:, 0, :], 0) * inv_n
    ms2 = jnp.sum(stf[:, 1, :], 0) * inv_n
    var = ms2 - m * m
    sc = g_ref[...][0] * lax.rsqrt(var + 1e-5)
    sh = b_ref[...][0] - m * sc
    h2 = jnp.maximum(z_ref[...] * sc + sh, 0.0)
    o = jnp.dot(h2, w3_ref[...].T, preferred_element_type=jnp.float32)
    o_ref[...] = jax.nn.sigmoid(o + b3_ref[...])

  return pl.pallas_call(
      body,
      grid=(t,),
      in_specs=[
          pl.BlockSpec((_BR, ho), lambda i: (i, 0)),
          pl.BlockSpec((t, 2, ho), lambda i: (0, 0, 0)),
          pl.BlockSpec((1, ho), lambda i: (0, 0)),
          pl.BlockSpec((1, ho), lambda i: (0, 0)),
          pl.BlockSpec((8, ho), lambda i: (0, 0)),
          pl.BlockSpec((1, 8), lambda i: (0, 0)),
      ],
      out_specs=pl.BlockSpec((_BR, 8), lambda i: (i, 0)),
      out_shape=jax.ShapeDtypeStruct((n, 8), jnp.float32),
  )(z, st, g.reshape(1, ho), beta.reshape(1, ho), w3p, b3p)


@functools.cache
def _get_sc_kernel():
  return _sc_layer_kernel()


def _pad_edges(ei, n_src, n_dst):
  padn = _EP - _E
  pad_src = (jnp.arange(padn, dtype=jnp.int32) * 8) % n_src
  pad_dst = jnp.full((padn,), n_dst, jnp.int32)
  return (jnp.concatenate([ei[0], pad_src]),
          jnp.concatenate([ei[1], pad_dst]))


def _pad_cols(a, w):
  pad = [(0, 0)] * (a.ndim - 1) + [(0, w - a.shape[-1])]
  return jnp.pad(a, pad)


def kernel(x_customer, x_transaction, x_device, x_email, x_address,
           ei_makes, ei_used_in, ei_linked_to, ei_located_at,
           ei_rev_makes, ei_rev_used_in, ei_rev_linked_to,
           ei_rev_located_at, params):
  xs = {"customer": x_customer, "transaction": x_transaction,
        "device": x_device, "email": x_email, "address": x_address}
  eis = {"makes": ei_makes, "used_in": ei_used_in,
         "linked_to": ei_linked_to, "located_at": ei_located_at,
         "rev_makes": ei_rev_makes, "rev_used_in": ei_rev_used_in,
         "rev_linked_to": ei_rev_linked_to,
         "rev_located_at": ei_rev_located_at}

  marker = jnp.zeros((_W,), jnp.float32).at[64].set(1.0)

  def bn128(g, beta):
    return _pad_cols(g, _W), _pad_cols(beta, _W) + marker

  h = {}
  for nt in _NT:
    pp = params["in_proj"][nt]
    z, st = _proj(xs[nt], _pad_cols(pp["W"].T, _W).T, _pad_cols(pp["b"], _W))
    g128, beta128 = bn128(pp["g"], pp["beta"])
    h[nt] = _apply_bn_relu(z, st, g128, beta128, _NN[nt])

  eip = [_pad_edges(eis[r], _NN[s], _NN[d]) for (r, s, d) in _RELS]
  sc = _get_sc_kernel()

  for layer in ["1", "2"]:
    outs = sc(*[h[nt] for nt in _NT],
              *[e[0] for e in eip], *[e[1] for e in eip])
    cv = params["conv" + layer]
    bn = params["bn" + layer]
    newh = {}
    for nt in _NT:
      ridx = [i for i, (_, _, d) in enumerate(_RELS) if d == nt]
      rnames = [_RELS[i][0] for i in ridx]

      def wpad(w):
        return jnp.pad(w, ((0, _W - _H), (0, _W - _H)))

      wl_st = jnp.stack([wpad(cv[r]["Wl"]) for r in rnames])
      wr_eff = wpad(sum(cv[r]["Wr"] for r in rnames))
      bl_eff = _pad_cols(sum(cv[r]["bl"] for r in rnames), _W)
      aggs = [outs[i] for i in ridx]
      z, st = _combine(h[nt], wr_eff, bl_eff, wl_st, aggs, _NN[nt])
      g128, beta128 = bn128(bn[nt]["g"], bn[nt]["b"])
      newh[nt] = _apply_bn_relu(z, st, g128, beta128, _NN[nt])
    h = newh

  c = params["cls"]
  nt_n = _NN["transaction"]
  z1, st1 = _proj(h["transaction"], _pad_cols(c["W1"], _W), c["b1"])
  h1 = _apply_bn_relu(z1, st1, c["g1"], c["beta1"], nt_n)
  z2, st2 = _proj(h1, c["W2"], c["b2"])
  w3p = jnp.zeros((8, c["W3"].shape[1]), jnp.float32).at[0].set(c["W3"][0])
  b3p = jnp.zeros((1, 8), jnp.float32).at[0, 0].set(c["b3"][0])
  out8 = _apply_bn_relu_sig(z2, st2, c["g2"], c["beta2"], w3p, b3p, nt_n)
  return out8[:, 0]


# speedup vs baseline: 2.1410x; 2.1410x over previous
import functools

import jax
import jax.numpy as jnp
from jax import lax
from jax.experimental import pallas as pl
from jax.experimental.pallas import tpu as pltpu
from jax.experimental.pallas import tpu_sc as plsc

_NT = ["customer", "transaction", "device", "email", "address"]
_NN = {"customer": 50000, "transaction": 100000, "device": 10000,
       "email": 10000, "address": 10000}
_H = 64
_W = 128
_E = 150000
_RELS = [
    ("makes", "customer", "transaction"),
    ("used_in", "device", "transaction"),
    ("linked_to", "email", "transaction"),
    ("located_at", "address", "transaction"),
    ("rev_makes", "transaction", "customer"),
    ("rev_used_in", "transaction", "device"),
    ("rev_linked_to", "transaction", "email"),
    ("rev_located_at", "transaction", "address"),
]

_NSUB = 16
_EPT = 9472
_EP = _NSUB * _EPT
_NBLK = _EPT // 128
_CHUNK = {"transaction": (8448, 6, 101376),
          "customer": (8448, 3, 50688),
          "device": (5120, 1, 10240),
          "email": (5120, 1, 10240),
          "address": (5120, 1, 10240)}
_ACC_ROWS = 8464
_ZROWS = 64


_EW = 2368
_NWIN = _EPT // _EW


def _sc_layer_kernel():
  mesh = plsc.VectorSubcoreMesh(core_axis_name="c", subcore_axis_name="s")
  out_type = [jax.ShapeDtypeStruct((_CHUNK[d][2], _W), jnp.float32)
              for (_, _, d) in _RELS]

  scratch = [
      pltpu.VMEM_SHARED((_ACC_ROWS, _W), jnp.float32),
      pltpu.VMEM((_EW,), jnp.int32),
      pltpu.VMEM((_EW,), jnp.int32),
      pltpu.VMEM((_EPT + 256,), jnp.int32),
      pltpu.VMEM((_EPT + 256,), jnp.int32),
      pltpu.VMEM((_NBLK, 128), jnp.int32),
      pltpu.VMEM((128, _W), jnp.float32),
      pltpu.VMEM((_ZROWS, _W), jnp.float32),
      pltpu.SemaphoreType.DMA,
  ]

  def body(*refs):
    h_refs = refs[0:5]
    src_refs = refs[5:13]
    dst_refs = refs[13:21]
    out_refs = refs[21:29]
    (acc_sh, srcw, dstw, psrc, pldst, pdst, rows, zb, sem) = refs[29:]

    s_idx = lax.axis_index("s")
    c_idx = lax.axis_index("c")

    zf = jnp.zeros((16,), jnp.float32)

    def fill_zb(r, _):
      for l in range(_W // 16):
        zb[r, pl.ds(l * 16, 16)] = zf
      return 0
    lax.fori_loop(0, _ZROWS, fill_zb, 0)

    h_by_nt = dict(zip(_NT, h_refs))

    for ri, (_, s_nt, d_nt) in enumerate(_RELS):
      cr, m, _ = _CHUNK[d_nt]
      h_ref = h_by_nt[s_nt]
      agg_ref = out_refs[ri]
      rpt = cr // _NSUB
      share0 = s_idx * rpt
      stripe0 = s_idx * _EPT

      def chunk(k, _):
        lo = (2 * k + c_idx) * cr

        nfull, rem = divmod(rpt, _ZROWS)
        for z in range(nfull):
          pltpu.sync_copy(zb, acc_sh.at[pl.ds(share0 + z * _ZROWS, _ZROWS)])
        if rem:
          pltpu.sync_copy(zb.at[pl.ds(0, rem)],
                          acc_sh.at[pl.ds(share0 + nfull * _ZROWS, rem)])

        def cvec_outer(i, off):
          def cvec(i, off):
            d = dstw[pl.ds(i * 16, 16)]
            s = srcw[pl.ds(i * 16, 16)]
            inr = (d >= lo) & (d < lo + cr)
            inc = jnp.where(inr, jnp.int32(1), jnp.int32(0))
            pos = jnp.where(inr, plsc.cumsum(inc) - 1 + off, _EPT + 240)
            plsc.store_scatter(psrc, [pos], s)
            plsc.store_scatter(pldst, [pos], d - lo)
            return off + jnp.sum(inc)
          return lax.fori_loop(0, _EW // 16, cvec, off)

        def win(w, off):
          pltpu.sync_copy(src_refs[ri].at[pl.ds(stripe0 + w * _EW, _EW)],
                          srcw)
          pltpu.sync_copy(dst_refs[ri].at[pl.ds(stripe0 + w * _EW, _EW)],
                          dstw)
          return cvec_outer(w, off)
        off = lax.fori_loop(0, _NWIN, win, jnp.int32(0))

        nblk = (off + 127) // 128

        for j in range(8):
          pos = off + j * 16 + lax.iota(jnp.int32, 16)
          plsc.store_scatter(psrc, [pos], jnp.zeros((16,), jnp.int32))
          plsc.store_scatter(pldst, [pos], jnp.full((16,), cr, jnp.int32))

        plsc.subcore_barrier()

        def blk(b, _):
          @pl.when(b < nblk)
          def _do():
            for l in range(8):
              pdst[b, pl.ds(l * 16, 16)] = pldst[pl.ds(b * 128 + l * 16, 16)]
            pltpu.async_copy(h_ref.at[psrc.at[pl.ds(b * 128, 128)]],
                             rows, sem).wait()
            pltpu.sync_copy(rows, acc_sh.at[pdst.at[b]], add=True)
          return 0
        lax.fori_loop(0, _NBLK, blk, 0)
        plsc.subcore_barrier()

        pltpu.sync_copy(acc_sh.at[pl.ds(share0, rpt)],
                        agg_ref.at[pl.ds(lo + share0, rpt)])
        return 0

      lax.fori_loop(0, m, chunk, 0)

  return pl.kernel(
      body, out_type=out_type, mesh=mesh, scratch_types=scratch,
      compiler_params=pltpu.CompilerParams(needs_layout_passes=False))


_BR = 1000


def _proj(x, w, b):
  n, kd = x.shape
  ho = w.shape[0]
  t = n // _BR

  def body(x_ref, w_ref, b_ref, z_ref, st_ref):
    z = jnp.dot(x_ref[...], w_ref[...].T,
                preferred_element_type=jnp.float32) + b_ref[...]
    z_ref[...] = z
    st_ref[...] = jnp.stack([jnp.sum(z, 0), jnp.sum(z * z, 0)])[None]

  return pl.pallas_call(
      body,
      grid=(t,),
      in_specs=[
          pl.BlockSpec((_BR, kd), lambda i: (i, 0)),
          pl.BlockSpec((ho, kd), lambda i: (0, 0)),
          pl.BlockSpec((1, ho), lambda i: (0, 0)),
      ],
      out_specs=[
          pl.BlockSpec((_BR, ho), lambda i: (i, 0)),
          pl.BlockSpec((1, 2, ho), lambda i: (i, 0, 0)),
      ],
      out_shape=[
          jax.ShapeDtypeStruct((n, ho), jnp.float32),
          jax.ShapeDtypeStruct((t, 2, ho), jnp.float32),
      ],
  )(x, w, b.reshape(1, ho))


def _combine(h, wr_eff, bl_eff, wl_st, aggs, n):
  nrel = wl_st.shape[0]
  t = n // _BR

  def body(*refs):
    h_ref, wr_ref, bl_ref, wl_ref = refs[0:4]
    agg_refs = refs[4:4 + nrel]
    z_ref, st_ref = refs[4 + nrel:]
    z = jnp.dot(h_ref[...], wr_ref[...].T,
                preferred_element_type=jnp.float32) + bl_ref[...]
    for r in range(nrel):
      a = agg_refs[r][...]
      cnt = jnp.maximum(a[:, 64:65], 1.0)
      mean = a / cnt
      z = z + jnp.dot(mean, wl_ref[r].T, preferred_element_type=jnp.float32)
    z_ref[...] = z
    st_ref[...] = jnp.stack([jnp.sum(z, 0), jnp.sum(z * z, 0)])[None]

  in_specs = [
      pl.BlockSpec((_BR, _W), lambda i: (i, 0)),
      pl.BlockSpec((_W, _W), lambda i: (0, 0)),
      pl.BlockSpec((1, _W), lambda i: (0, 0)),
      pl.BlockSpec((nrel, _W, _W), lambda i: (0, 0, 0)),
  ]
  in_specs += [pl.BlockSpec((_BR, _W), lambda i: (i, 0))] * nrel

  return pl.pallas_call(
      body,
      grid=(t,),
      in_specs=in_specs,
      out_specs=[
          pl.BlockSpec((_BR, _W), lambda i: (i, 0)),
          pl.BlockSpec((1, 2, _W), lambda i: (i, 0, 0)),
      ],
      out_shape=[
          jax.ShapeDtypeStruct((n, _W), jnp.float32),
          jax.ShapeDtypeStruct((t, 2, _W), jnp.float32),
      ],
  )(h, wr_eff, bl_eff.reshape(1, _W), wl_st, *aggs)


def _apply_bn_relu(z, st, g, beta, n):
  ho = z.shape[1]
  t = n // _BR

  def body(z_ref, st_ref, g_ref, b_ref, h_ref):
    stf = st_ref[...]
    inv_n = 1.0 / n
    m = jnp.sum(stf[:, 0, :], 0) * inv_n
    ms2 = jnp.sum(stf[:, 1, :], 0) * inv_n
    var = ms2 - m * m
    sc = g_ref[...][0] * lax.rsqrt(var + 1e-5)
    sh = b_ref[...][0] - m * sc
    h_ref[...] = jnp.maximum(z_ref[...] * sc + sh, 0.0)

  return pl.pallas_call(
      body,
      grid=(t,),
      in_specs=[
          pl.BlockSpec((_BR, ho), lambda i: (i, 0)),
          pl.BlockSpec((t, 2, ho), lambda i: (0, 0, 0)),
          pl.BlockSpec((1, ho), lambda i: (0, 0)),
          pl.BlockSpec((1, ho), lambda i: (0, 0)),
      ],
      out_specs=pl.BlockSpec((_BR, ho), lambda i: (i, 0)),
      out_shape=jax.ShapeDtypeStruct((n, ho), jnp.float32),
  )(z, st, g.reshape(1, ho), beta.reshape(1, ho))


def _apply_bn_relu_sig(z, st, g, beta, w3p, b3p, n):
  ho = z.shape[1]
  t = n // _BR

  def body(z_ref, st_ref, g_ref, b_ref, w3_ref, b3_ref, o_ref):
    stf = st_ref[...]
    inv_n = 1.0 / n
    m = jnp.sum(stf[:, 0, :], 0) * inv_n
    ms2 = jnp.sum(stf[:, 1, :], 0) * inv_n
    var = ms2 - m * m
    sc = g_ref[...][0] * lax.rsqrt(var + 1e-5)
    sh = b_ref[...][0] - m * sc
    h2 = jnp.maximum(z_ref[...] * sc + sh, 0.0)
    o = jnp.dot(h2, w3_ref[...].T, preferred_element_type=jnp.float32)
    o_ref[...] = jax.nn.sigmoid(o + b3_ref[...])

  return pl.pallas_call(
      body,
      grid=(t,),
      in_specs=[
          pl.BlockSpec((_BR, ho), lambda i: (i, 0)),
          pl.BlockSpec((t, 2, ho), lambda i: (0, 0, 0)),
          pl.BlockSpec((1, ho), lambda i: (0, 0)),
          pl.BlockSpec((1, ho), lambda i: (0, 0)),
          pl.BlockSpec((8, ho), lambda i: (0, 0)),
          pl.BlockSpec((1, 8), lambda i: (0, 0)),
      ],
      out_specs=pl.BlockSpec((_BR, 8), lambda i: (i, 0)),
      out_shape=jax.ShapeDtypeStruct((n, 8), jnp.float32),
  )(z, st, g.reshape(1, ho), beta.reshape(1, ho), w3p, b3p)


@functools.cache
def _get_sc_kernel():
  return _sc_layer_kernel()


def _pad_edges(ei, n_src, n_dst):
  padn = _EP - _E
  pad_src = (jnp.arange(padn, dtype=jnp.int32) * 8) % n_src
  pad_dst = jnp.full((padn,), n_dst, jnp.int32)
  return (jnp.concatenate([ei[0], pad_src]),
          jnp.concatenate([ei[1], pad_dst]))


def _pad_cols(a, w):
  pad = [(0, 0)] * (a.ndim - 1) + [(0, w - a.shape[-1])]
  return jnp.pad(a, pad)


def kernel(x_customer, x_transaction, x_device, x_email, x_address,
           ei_makes, ei_used_in, ei_linked_to, ei_located_at,
           ei_rev_makes, ei_rev_used_in, ei_rev_linked_to,
           ei_rev_located_at, params):
  xs = {"customer": x_customer, "transaction": x_transaction,
        "device": x_device, "email": x_email, "address": x_address}
  eis = {"makes": ei_makes, "used_in": ei_used_in,
         "linked_to": ei_linked_to, "located_at": ei_located_at,
         "rev_makes": ei_rev_makes, "rev_used_in": ei_rev_used_in,
         "rev_linked_to": ei_rev_linked_to,
         "rev_located_at": ei_rev_located_at}

  marker = jnp.zeros((_W,), jnp.float32).at[64].set(1.0)

  def bn128(g, beta):
    return _pad_cols(g, _W), _pad_cols(beta, _W) + marker

  h = {}
  for nt in _NT:
    pp = params["in_proj"][nt]
    z, st = _proj(xs[nt], _pad_cols(pp["W"].T, _W).T, _pad_cols(pp["b"], _W))
    g128, beta128 = bn128(pp["g"], pp["beta"])
    h[nt] = _apply_bn_relu(z, st, g128, beta128, _NN[nt])

  eip = [_pad_edges(eis[r], _NN[s], _NN[d]) for (r, s, d) in _RELS]
  sc = _get_sc_kernel()

  for layer in ["1", "2"]:
    outs = sc(*[h[nt] for nt in _NT],
              *[e[0] for e in eip], *[e[1] for e in eip])
    cv = params["conv" + layer]
    bn = params["bn" + layer]
    newh = {}
    for nt in _NT:
      ridx = [i for i, (_, _, d) in enumerate(_RELS) if d == nt]
      rnames = [_RELS[i][0] for i in ridx]

      def wpad(w):
        return jnp.pad(w, ((0, _W - _H), (0, _W - _H)))

      wl_st = jnp.stack([wpad(cv[r]["Wl"]) for r in rnames])
      wr_eff = wpad(sum(cv[r]["Wr"] for r in rnames))
      bl_eff = _pad_cols(sum(cv[r]["bl"] for r in rnames), _W)
      aggs = [outs[i] for i in ridx]
      z, st = _combine(h[nt], wr_eff, bl_eff, wl_st, aggs, _NN[nt])
      g128, beta128 = bn128(bn[nt]["g"], bn[nt]["b"])
      newh[nt] = _apply_bn_relu(z, st, g128, beta128, _NN[nt])
    h = newh

  c = params["cls"]
  nt_n = _NN["transaction"]
  z1, st1 = _proj(h["transaction"], _pad_cols(c["W1"], _W), c["b1"])
  h1 = _apply_bn_relu(z1, st1, c["g1"], c["beta1"], nt_n)
  z2, st2 = _proj(h1, c["W2"], c["b2"])
  w3p = jnp.zeros((8, c["W3"].shape[1]), jnp.float32).at[0].set(c["W3"][0])
  b3p = jnp.zeros((1, 8), jnp.float32).at[0, 0].set(c["b3"][0])
  out8 = _apply_bn_relu_sig(z2, st2, c["g2"], c["beta2"], w3p, b3p, nt_n)
  return out8[:, 0]


# speedup vs baseline: 2.3737x; 1.1087x over previous
import functools

import jax
import jax.numpy as jnp
from jax import lax
from jax.experimental import pallas as pl
from jax.experimental.pallas import tpu as pltpu
from jax.experimental.pallas import tpu_sc as plsc

_NT = ["customer", "transaction", "device", "email", "address"]
_NN = {"customer": 50000, "transaction": 100000, "device": 10000,
       "email": 10000, "address": 10000}
_H = 64
_W = 128
_E = 150000
_RELS = [
    ("makes", "customer", "transaction"),
    ("used_in", "device", "transaction"),
    ("linked_to", "email", "transaction"),
    ("located_at", "address", "transaction"),
    ("rev_makes", "transaction", "customer"),
    ("rev_used_in", "transaction", "device"),
    ("rev_linked_to", "transaction", "email"),
    ("rev_located_at", "transaction", "address"),
]

_NSUB = 16
_EPT = 9472
_EP = _NSUB * _EPT
_NBLK = _EPT // 128
_CHUNK = {"transaction": (8448, 6, 101376),
          "customer": (8448, 3, 50688),
          "device": (5120, 1, 10240),
          "email": (5120, 1, 10240),
          "address": (5120, 1, 10240)}
_ACC_ROWS = 8464
_ZROWS = 64


_EW = 2368
_NWIN = _EPT // _EW
_PCAP = _EPT + 256


def _sc_bin_kernel():
  mesh = plsc.VectorSubcoreMesh(core_axis_name="c", subcore_axis_name="s")
  out_type = []
  for (_, _, d) in _RELS:
    m = _CHUNK[d][1]
    out_type += [
        jax.ShapeDtypeStruct((2 * m * _NSUB * _PCAP,), jnp.int32),
        jax.ShapeDtypeStruct((2 * m * _NSUB * _PCAP,), jnp.int32),
        jax.ShapeDtypeStruct((2 * m * _NSUB * 16,), jnp.int32),
    ]

  scratch = [
      pltpu.VMEM((_EW,), jnp.int32),
      pltpu.VMEM((_EW,), jnp.int32),
      pltpu.VMEM((_PCAP,), jnp.int32),
      pltpu.VMEM((_PCAP,), jnp.int32),
      pltpu.VMEM((16,), jnp.int32),
  ]

  def body(*refs):
    src_refs = refs[0:8]
    dst_refs = refs[8:16]
    out_refs = refs[16:40]
    (srcw, dstw, psrc, pldst, cntv) = refs[40:]

    s_idx = lax.axis_index("s")
    c_idx = lax.axis_index("c")
    zi = jnp.zeros((16,), jnp.int32)
    lane = lax.iota(jnp.int32, 16)

    for ri, (_, _, d_nt) in enumerate(_RELS):
      cr, m, _ = _CHUNK[d_nt]
      ps_ref, pd_ref, cn_ref = out_refs[3 * ri:3 * ri + 3]
      stripe0 = s_idx * _EPT

      def chunk(k, _):
        g = 2 * k + c_idx
        lo = g * cr

        def cvec_outer(i, off):
          def cvec(i, off):
            d = dstw[pl.ds(i * 16, 16)]
            s = srcw[pl.ds(i * 16, 16)]
            inr = (d >= lo) & (d < lo + cr)
            inc = jnp.where(inr, jnp.int32(1), jnp.int32(0))
            pos = jnp.where(inr, plsc.cumsum(inc) - 1 + off, _EPT + 240)
            plsc.store_scatter(psrc, [pos], s)
            plsc.store_scatter(pldst, [pos], d - lo)
            return off + jnp.sum(inc)
          return lax.fori_loop(0, _EW // 16, cvec, off)

        def win(w, off):
          pltpu.sync_copy(src_refs[ri].at[pl.ds(stripe0 + w * _EW, _EW)],
                          srcw)
          pltpu.sync_copy(dst_refs[ri].at[pl.ds(stripe0 + w * _EW, _EW)],
                          dstw)
          return cvec_outer(w, off)
        off = lax.fori_loop(0, _NWIN, win, jnp.int32(0))

        for j in range(8):
          pos = off + j * 16 + lane
          plsc.store_scatter(psrc, [pos], zi)
          plsc.store_scatter(pldst, [pos], jnp.full((16,), cr, jnp.int32))

        base = (g * _NSUB + s_idx) * _PCAP
        pltpu.sync_copy(psrc, ps_ref.at[pl.ds(base, _PCAP)])
        pltpu.sync_copy(pldst, pd_ref.at[pl.ds(base, _PCAP)])
        cntv[pl.ds(0, 16)] = jnp.where(lane == 0, zi + off, zi)
        pltpu.sync_copy(cntv, cn_ref.at[pl.ds((g * _NSUB + s_idx) * 16, 16)])
        return 0

      lax.fori_loop(0, m, chunk, 0)

  return pl.kernel(
      body, out_type=out_type, mesh=mesh, scratch_types=scratch,
      compiler_params=pltpu.CompilerParams(needs_layout_passes=False))


def _sc_agg_kernel():
  mesh = plsc.VectorSubcoreMesh(core_axis_name="c", subcore_axis_name="s")
  out_type = [jax.ShapeDtypeStruct((_CHUNK[d][2], _W), jnp.float32)
              for (_, _, d) in _RELS]

  scratch = [
      pltpu.VMEM_SHARED((_ACC_ROWS, _W), jnp.float32),
      pltpu.VMEM((_PCAP,), jnp.int32),
      pltpu.VMEM((_PCAP,), jnp.int32),
      pltpu.VMEM((16,), jnp.int32),
      pltpu.VMEM((_NBLK + 2, 128), jnp.int32),
      pltpu.VMEM((128, _W), jnp.float32),
      pltpu.VMEM((_ZROWS, _W), jnp.float32),
      pltpu.SemaphoreType.DMA,
  ]

  def body(*refs):
    h_refs = refs[0:5]
    pk_refs = refs[5:29]
    out_refs = refs[29:37]
    (acc_sh, psrc, pldst, cntv, pdst, rows, zb, sem) = refs[37:]

    s_idx = lax.axis_index("s")
    c_idx = lax.axis_index("c")

    zf = jnp.zeros((16,), jnp.float32)

    def fill_zb(r, _):
      for l in range(_W // 16):
        zb[r, pl.ds(l * 16, 16)] = zf
      return 0
    lax.fori_loop(0, _ZROWS, fill_zb, 0)

    h_by_nt = dict(zip(_NT, h_refs))

    for ri, (_, s_nt, d_nt) in enumerate(_RELS):
      cr, m, _ = _CHUNK[d_nt]
      h_ref = h_by_nt[s_nt]
      ps_ref, pd_ref, cn_ref = pk_refs[3 * ri:3 * ri + 3]
      agg_ref = out_refs[ri]
      rpt = cr // _NSUB
      share0 = s_idx * rpt

      def chunk(k, _):
        g = 2 * k + c_idx
        lo = g * cr

        nfull, rem = divmod(rpt, _ZROWS)
        for z in range(nfull):
          pltpu.sync_copy(zb, acc_sh.at[pl.ds(share0 + z * _ZROWS, _ZROWS)])
        if rem:
          pltpu.sync_copy(zb.at[pl.ds(0, rem)],
                          acc_sh.at[pl.ds(share0 + nfull * _ZROWS, rem)])

        base = (g * _NSUB + s_idx) * _PCAP
        pltpu.sync_copy(ps_ref.at[pl.ds(base, _PCAP)], psrc)
        pltpu.sync_copy(pd_ref.at[pl.ds(base, _PCAP)], pldst)
        pltpu.sync_copy(cn_ref.at[pl.ds((g * _NSUB + s_idx) * 16, 16)], cntv)
        off = jnp.sum(cntv[pl.ds(0, 16)])
        nblk = (off + 127) // 128

        plsc.subcore_barrier()

        def blk(b, _):
          @pl.when(b < nblk)
          def _do():
            for l in range(8):
              pdst[b, pl.ds(l * 16, 16)] = pldst[pl.ds(b * 128 + l * 16, 16)]
            pltpu.async_copy(h_ref.at[psrc.at[pl.ds(b * 128, 128)]],
                             rows, sem).wait()
            pltpu.sync_copy(rows, acc_sh.at[pdst.at[b]], add=True)
          return 0
        lax.fori_loop(0, _NBLK + 1, blk, 0)
        plsc.subcore_barrier()

        pltpu.sync_copy(acc_sh.at[pl.ds(share0, rpt)],
                        agg_ref.at[pl.ds(lo + share0, rpt)])
        return 0

      lax.fori_loop(0, m, chunk, 0)

  return pl.kernel(
      body, out_type=out_type, mesh=mesh, scratch_types=scratch,
      compiler_params=pltpu.CompilerParams(needs_layout_passes=False))


_BR = 1000


def _proj(x, w, b):
  n, kd = x.shape
  ho = w.shape[0]
  t = n // _BR

  def body(x_ref, w_ref, b_ref, z_ref, st_ref):
    z = jnp.dot(x_ref[...], w_ref[...].T,
                preferred_element_type=jnp.float32) + b_ref[...]
    z_ref[...] = z
    st_ref[...] = jnp.stack([jnp.sum(z, 0), jnp.sum(z * z, 0)])[None]

  return pl.pallas_call(
      body,
      grid=(t,),
      in_specs=[
          pl.BlockSpec((_BR, kd), lambda i: (i, 0)),
          pl.BlockSpec((ho, kd), lambda i: (0, 0)),
          pl.BlockSpec((1, ho), lambda i: (0, 0)),
      ],
      out_specs=[
          pl.BlockSpec((_BR, ho), lambda i: (i, 0)),
          pl.BlockSpec((1, 2, ho), lambda i: (i, 0, 0)),
      ],
      out_shape=[
          jax.ShapeDtypeStruct((n, ho), jnp.float32),
          jax.ShapeDtypeStruct((t, 2, ho), jnp.float32),
      ],
  )(x, w, b.reshape(1, ho))


def _combine(h, wr_eff, bl_eff, wl_st, aggs, n):
  nrel = wl_st.shape[0]
  t = n // _BR

  def body(*refs):
    h_ref, wr_ref, bl_ref, wl_ref = refs[0:4]
    agg_refs = refs[4:4 + nrel]
    z_ref, st_ref = refs[4 + nrel:]
    z = jnp.dot(h_ref[...], wr_ref[...].T,
                preferred_element_type=jnp.float32) + bl_ref[...]
    for r in range(nrel):
      a = agg_refs[r][...]
      cnt = jnp.maximum(a[:, 64:65], 1.0)
      mean = a / cnt
      z = z + jnp.dot(mean, wl_ref[r].T, preferred_element_type=jnp.float32)
    z_ref[...] = z
    st_ref[...] = jnp.stack([jnp.sum(z, 0), jnp.sum(z * z, 0)])[None]

  in_specs = [
      pl.BlockSpec((_BR, _W), lambda i: (i, 0)),
      pl.BlockSpec((_W, _W), lambda i: (0, 0)),
      pl.BlockSpec((1, _W), lambda i: (0, 0)),
      pl.BlockSpec((nrel, _W, _W), lambda i: (0, 0, 0)),
  ]
  in_specs += [pl.BlockSpec((_BR, _W), lambda i: (i, 0))] * nrel

  return pl.pallas_call(
      body,
      grid=(t,),
      in_specs=in_specs,
      out_specs=[
          pl.BlockSpec((_BR, _W), lambda i: (i, 0)),
          pl.BlockSpec((1, 2, _W), lambda i: (i, 0, 0)),
      ],
      out_shape=[
          jax.ShapeDtypeStruct((n, _W), jnp.float32),
          jax.ShapeDtypeStruct((t, 2, _W), jnp.float32),
      ],
  )(h, wr_eff, bl_eff.reshape(1, _W), wl_st, *aggs)


def _apply_bn_relu(z, st, g, beta, n):
  ho = z.shape[1]
  t = n // _BR

  def body(z_ref, st_ref, g_ref, b_ref, h_ref):
    stf = st_ref[...]
    inv_n = 1.0 / n
    m = jnp.sum(stf[:, 0, :], 0) * inv_n
    ms2 = jnp.sum(stf[:, 1, :], 0) * inv_n
    var = ms2 - m * m
    sc = g_ref[...][0] * lax.rsqrt(var + 1e-5)
    sh = b_ref[...][0] - m * sc
    h_ref[...] = jnp.maximum(z_ref[...] * sc + sh, 0.0)

  return pl.pallas_call(
      body,
      grid=(t,),
      in_specs=[
          pl.BlockSpec((_BR, ho), lambda i: (i, 0)),
          pl.BlockSpec((t, 2, ho), lambda i: (0, 0, 0)),
          pl.BlockSpec((1, ho), lambda i: (0, 0)),
          pl.BlockSpec((1, ho), lambda i: (0, 0)),
      ],
      out_specs=pl.BlockSpec((_BR, ho), lambda i: (i, 0)),
      out_shape=jax.ShapeDtypeStruct((n, ho), jnp.float32),
  )(z, st, g.reshape(1, ho), beta.reshape(1, ho))


def _apply_bn_relu_sig(z, st, g, beta, w3p, b3p, n):
  ho = z.shape[1]
  t = n // _BR

  def body(z_ref, st_ref, g_ref, b_ref, w3_ref, b3_ref, o_ref):
    stf = st_ref[...]
    inv_n = 1.0 / n
    m = jnp.sum(stf[:, 0, :], 0) * inv_n
    ms2 = jnp.sum(stf[:, 1, :], 0) * inv_n
    var = ms2 - m * m
    sc = g_ref[...][0] * lax.rsqrt(var + 1e-5)
    sh = b_ref[...][0] - m * sc
    h2 = jnp.maximum(z_ref[...] * sc + sh, 0.0)
    o = jnp.dot(h2, w3_ref[...].T, preferred_element_type=jnp.float32)
    o_ref[...] = jax.nn.sigmoid(o + b3_ref[...])

  return pl.pallas_call(
      body,
      grid=(t,),
      in_specs=[
          pl.BlockSpec((_BR, ho), lambda i: (i, 0)),
          pl.BlockSpec((t, 2, ho), lambda i: (0, 0, 0)),
          pl.BlockSpec((1, ho), lambda i: (0, 0)),
          pl.BlockSpec((1, ho), lambda i: (0, 0)),
          pl.BlockSpec((8, ho), lambda i: (0, 0)),
          pl.BlockSpec((1, 8), lambda i: (0, 0)),
      ],
      out_specs=pl.BlockSpec((_BR, 8), lambda i: (i, 0)),
      out_shape=jax.ShapeDtypeStruct((n, 8), jnp.float32),
  )(z, st, g.reshape(1, ho), beta.reshape(1, ho), w3p, b3p)


@functools.cache
def _get_sc_bin():
  return _sc_bin_kernel()


@functools.cache
def _get_sc_agg():
  return _sc_agg_kernel()


def _pad_edges(ei, n_src, n_dst):
  padn = _EP - _E
  pad_src = (jnp.arange(padn, dtype=jnp.int32) * 8) % n_src
  pad_dst = jnp.full((padn,), n_dst, jnp.int32)
  return (jnp.concatenate([ei[0], pad_src]),
          jnp.concatenate([ei[1], pad_dst]))


def _pad_cols(a, w):
  pad = [(0, 0)] * (a.ndim - 1) + [(0, w - a.shape[-1])]
  return jnp.pad(a, pad)


def kernel(x_customer, x_transaction, x_device, x_email, x_address,
           ei_makes, ei_used_in, ei_linked_to, ei_located_at,
           ei_rev_makes, ei_rev_used_in, ei_rev_linked_to,
           ei_rev_located_at, params):
  xs = {"customer": x_customer, "transaction": x_transaction,
        "device": x_device, "email": x_email, "address": x_address}
  eis = {"makes": ei_makes, "used_in": ei_used_in,
         "linked_to": ei_linked_to, "located_at": ei_located_at,
         "rev_makes": ei_rev_makes, "rev_used_in": ei_rev_used_in,
         "rev_linked_to": ei_rev_linked_to,
         "rev_located_at": ei_rev_located_at}

  marker = jnp.zeros((_W,), jnp.float32).at[64].set(1.0)

  def bn128(g, beta):
    return _pad_cols(g, _W), _pad_cols(beta, _W) + marker

  h = {}
  for nt in _NT:
    pp = params["in_proj"][nt]
    z, st = _proj(xs[nt], _pad_cols(pp["W"].T, _W).T, _pad_cols(pp["b"], _W))
    g128, beta128 = bn128(pp["g"], pp["beta"])
    h[nt] = _apply_bn_relu(z, st, g128, beta128, _NN[nt])

  eip = [_pad_edges(eis[r], _NN[s], _NN[d]) for (r, s, d) in _RELS]
  packed = _get_sc_bin()(*[e[0] for e in eip], *[e[1] for e in eip])
  agg = _get_sc_agg()

  for layer in ["1", "2"]:
    outs = agg(*[h[nt] for nt in _NT], *packed)
    cv = params["conv" + layer]
    bn = params["bn" + layer]
    newh = {}
    for nt in _NT:
      ridx = [i for i, (_, _, d) in enumerate(_RELS) if d == nt]
      rnames = [_RELS[i][0] for i in ridx]

      def wpad(w):
        return jnp.pad(w, ((0, _W - _H), (0, _W - _H)))

      wl_st = jnp.stack([wpad(cv[r]["Wl"]) for r in rnames])
      wr_eff = wpad(sum(cv[r]["Wr"] for r in rnames))
      bl_eff = _pad_cols(sum(cv[r]["bl"] for r in rnames), _W)
      aggs = [outs[i] for i in ridx]
      z, st = _combine(h[nt], wr_eff, bl_eff, wl_st, aggs, _NN[nt])
      g128, beta128 = bn128(bn[nt]["g"], bn[nt]["b"])
      newh[nt] = _apply_bn_relu(z, st, g128, beta128, _NN[nt])
    h = newh

  c = params["cls"]
  nt_n = _NN["transaction"]
  z1, st1 = _proj(h["transaction"], _pad_cols(c["W1"], _W), c["b1"])
  h1 = _apply_bn_relu(z1, st1, c["g1"], c["beta1"], nt_n)
  z2, st2 = _proj(h1, c["W2"], c["b2"])
  w3p = jnp.zeros((8, c["W3"].shape[1]), jnp.float32).at[0].set(c["W3"][0])
  b3p = jnp.zeros((1, 8), jnp.float32).at[0, 0].set(c["b3"][0])
  out8 = _apply_bn_relu_sig(z2, st2, c["g2"], c["beta2"], w3p, b3p, nt_n)
  return out8[:, 0]


# speedup vs baseline: 2.5815x; 1.0875x over previous
import functools

import jax
import jax.numpy as jnp
from jax import lax
from jax.experimental import pallas as pl
from jax.experimental.pallas import tpu as pltpu
from jax.experimental.pallas import tpu_sc as plsc

_NT = ["customer", "transaction", "device", "email", "address"]
_NN = {"customer": 50000, "transaction": 100000, "device": 10000,
       "email": 10000, "address": 10000}
_H = 64
_W = 128
_E = 150000
_RELS = [
    ("makes", "customer", "transaction"),
    ("used_in", "device", "transaction"),
    ("linked_to", "email", "transaction"),
    ("located_at", "address", "transaction"),
    ("rev_makes", "transaction", "customer"),
    ("rev_used_in", "transaction", "device"),
    ("rev_linked_to", "transaction", "email"),
    ("rev_located_at", "transaction", "address"),
]

_NSUB = 16
_EPT = 9472
_EP = _NSUB * _EPT
_NBLK = _EPT // 128
_CHUNK = {"transaction": (8448, 6, 101376),
          "customer": (8448, 3, 50688),
          "device": (5120, 1, 10240),
          "email": (5120, 1, 10240),
          "address": (5120, 1, 10240)}
_ACC_ROWS = 8464
_ZROWS = 64


_EW = 2368
_NWIN = _EPT // _EW
_PCAP = _EPT + 256


def _sc_bin_kernel():
  mesh = plsc.VectorSubcoreMesh(core_axis_name="c", subcore_axis_name="s")
  out_type = []
  for (_, _, d) in _RELS:
    m = _CHUNK[d][1]
    out_type += [
        jax.ShapeDtypeStruct((2 * m * _NSUB * _PCAP,), jnp.int32),
        jax.ShapeDtypeStruct((2 * m * _NSUB * _PCAP,), jnp.int32),
        jax.ShapeDtypeStruct((2 * m * _NSUB * 16,), jnp.int32),
    ]

  scratch = [
      pltpu.VMEM((_EW,), jnp.int32),
      pltpu.VMEM((_EW,), jnp.int32),
      pltpu.VMEM((_PCAP,), jnp.int32),
      pltpu.VMEM((_PCAP,), jnp.int32),
      pltpu.VMEM((16,), jnp.int32),
  ]

  def body(*refs):
    src_refs = refs[0:8]
    dst_refs = refs[8:16]
    out_refs = refs[16:40]
    (srcw, dstw, psrc, pldst, cntv) = refs[40:]

    s_idx = lax.axis_index("s")
    c_idx = lax.axis_index("c")
    zi = jnp.zeros((16,), jnp.int32)
    lane = lax.iota(jnp.int32, 16)

    for ri, (_, _, d_nt) in enumerate(_RELS):
      cr, m, _ = _CHUNK[d_nt]
      ps_ref, pd_ref, cn_ref = out_refs[3 * ri:3 * ri + 3]
      stripe0 = s_idx * _EPT

      def chunk(k, _):
        g = 2 * k + c_idx
        lo = g * cr

        def cvec_outer(i, off):
          def cvec(i, off):
            d = dstw[pl.ds(i * 16, 16)]
            s = srcw[pl.ds(i * 16, 16)]
            inr = (d >= lo) & (d < lo + cr)
            inc = jnp.where(inr, jnp.int32(1), jnp.int32(0))
            pos = jnp.where(inr, plsc.cumsum(inc) - 1 + off, _EPT + 240)
            plsc.store_scatter(psrc, [pos], s)
            plsc.store_scatter(pldst, [pos], d - lo)
            return off + jnp.sum(inc)
          return lax.fori_loop(0, _EW // 16, cvec, off)

        def win(w, off):
          pltpu.sync_copy(src_refs[ri].at[pl.ds(stripe0 + w * _EW, _EW)],
                          srcw)
          pltpu.sync_copy(dst_refs[ri].at[pl.ds(stripe0 + w * _EW, _EW)],
                          dstw)
          return cvec_outer(w, off)
        off = lax.fori_loop(0, _NWIN, win, jnp.int32(0))

        for j in range(8):
          pos = off + j * 16 + lane
          plsc.store_scatter(psrc, [pos], zi)
          plsc.store_scatter(pldst, [pos], jnp.full((16,), cr, jnp.int32))

        base = (g * _NSUB + s_idx) * _PCAP
        pltpu.sync_copy(psrc, ps_ref.at[pl.ds(base, _PCAP)])
        pltpu.sync_copy(pldst, pd_ref.at[pl.ds(base, _PCAP)])
        cntv[pl.ds(0, 16)] = jnp.where(lane == 0, zi + off, zi)
        pltpu.sync_copy(cntv, cn_ref.at[pl.ds((g * _NSUB + s_idx) * 16, 16)])
        return 0

      lax.fori_loop(0, m, chunk, 0)

  return pl.kernel(
      body, out_type=out_type, mesh=mesh, scratch_types=scratch,
      compiler_params=pltpu.CompilerParams(needs_layout_passes=False))


def _sc_agg_kernel():
  mesh = plsc.VectorSubcoreMesh(core_axis_name="c", subcore_axis_name="s")
  out_type = [jax.ShapeDtypeStruct((_CHUNK[d][2], _W), jnp.float32)
              for (_, _, d) in _RELS]

  scratch = [
      pltpu.VMEM_SHARED((_ACC_ROWS, _W), jnp.float32),
      pltpu.VMEM((_PCAP,), jnp.int32),
      pltpu.VMEM((_PCAP,), jnp.int32),
      pltpu.VMEM((16,), jnp.int32),
      pltpu.VMEM((2, 128), jnp.int32),
      pltpu.VMEM((128, _W), jnp.float32),
      pltpu.VMEM((128, _W), jnp.float32),
      pltpu.VMEM((_ZROWS, _W), jnp.float32),
      pltpu.SemaphoreType.DMA,
      pltpu.SemaphoreType.DMA,
  ]

  def body(*refs):
    h_refs = refs[0:5]
    pk_refs = refs[5:29]
    out_refs = refs[29:37]
    (acc_sh, psrc, pldst, cntv, pdst, rows0, rows1, zb,
     sem0, sem1) = refs[37:]

    s_idx = lax.axis_index("s")
    c_idx = lax.axis_index("c")

    zf = jnp.zeros((16,), jnp.float32)

    def fill_zb(r, _):
      for l in range(_W // 16):
        zb[r, pl.ds(l * 16, 16)] = zf
      return 0
    lax.fori_loop(0, _ZROWS, fill_zb, 0)

    h_by_nt = dict(zip(_NT, h_refs))

    for ri, (_, s_nt, d_nt) in enumerate(_RELS):
      cr, m, _ = _CHUNK[d_nt]
      h_ref = h_by_nt[s_nt]
      ps_ref, pd_ref, cn_ref = pk_refs[3 * ri:3 * ri + 3]
      agg_ref = out_refs[ri]
      rpt = cr // _NSUB
      share0 = s_idx * rpt

      def chunk(k, _):
        g = 2 * k + c_idx
        lo = g * cr

        nfull, rem = divmod(rpt, _ZROWS)
        for z in range(nfull):
          pltpu.sync_copy(zb, acc_sh.at[pl.ds(share0 + z * _ZROWS, _ZROWS)])
        if rem:
          pltpu.sync_copy(zb.at[pl.ds(0, rem)],
                          acc_sh.at[pl.ds(share0 + nfull * _ZROWS, rem)])

        base = (g * _NSUB + s_idx) * _PCAP
        pltpu.sync_copy(ps_ref.at[pl.ds(base, _PCAP)], psrc)
        pltpu.sync_copy(pd_ref.at[pl.ds(base, _PCAP)], pldst)
        pltpu.sync_copy(cn_ref.at[pl.ds((g * _NSUB + s_idx) * 16, 16)], cntv)
        off = jnp.sum(cntv[pl.ds(0, 16)])
        nblk = (off + 127) // 128

        plsc.subcore_barrier()

        def start(b, rows, sem):
          pltpu.async_copy(h_ref.at[psrc.at[pl.ds(b * 128, 128)]], rows, sem)

        def drain(b, j, rows, sem):
          for l in range(8):
            pdst[j, pl.ds(l * 16, 16)] = pldst[pl.ds(b * 128 + l * 16, 16)]
          pltpu.make_async_copy(h_ref.at[psrc.at[pl.ds(b * 128, 128)]],
                                rows, sem).wait()
          pltpu.sync_copy(rows, acc_sh.at[pdst.at[j]], add=True)

        @pl.when(nblk > 0)
        def _p0():
          start(0, rows0, sem0)

        def pair(g, _):
          b0 = 2 * g
          b1 = b0 + 1

          @pl.when(b1 < nblk)
          def _s1():
            start(b1, rows1, sem1)

          @pl.when(b0 < nblk)
          def _d0():
            drain(b0, 0, rows0, sem0)

          @pl.when(b1 + 1 < nblk)
          def _s2():
            start(b1 + 1, rows0, sem0)

          @pl.when(b1 < nblk)
          def _d1():
            drain(b1, 1, rows1, sem1)
          return 0
        lax.fori_loop(0, (_NBLK + 1) // 2, pair, 0)
        plsc.subcore_barrier()

        pltpu.sync_copy(acc_sh.at[pl.ds(share0, rpt)],
                        agg_ref.at[pl.ds(lo + share0, rpt)])
        return 0

      lax.fori_loop(0, m, chunk, 0)

  return pl.kernel(
      body, out_type=out_type, mesh=mesh, scratch_types=scratch,
      compiler_params=pltpu.CompilerParams(needs_layout_passes=False))


_BR = 1000


def _proj(x, w, b):
  n, kd = x.shape
  ho = w.shape[0]
  t = n // _BR

  def body(x_ref, w_ref, b_ref, z_ref, st_ref):
    z = jnp.dot(x_ref[...], w_ref[...].T,
                preferred_element_type=jnp.float32) + b_ref[...]
    z_ref[...] = z
    st_ref[...] = jnp.stack([jnp.sum(z, 0), jnp.sum(z * z, 0)])[None]

  return pl.pallas_call(
      body,
      grid=(t,),
      in_specs=[
          pl.BlockSpec((_BR, kd), lambda i: (i, 0)),
          pl.BlockSpec((ho, kd), lambda i: (0, 0)),
          pl.BlockSpec((1, ho), lambda i: (0, 0)),
      ],
      out_specs=[
          pl.BlockSpec((_BR, ho), lambda i: (i, 0)),
          pl.BlockSpec((1, 2, ho), lambda i: (i, 0, 0)),
      ],
      out_shape=[
          jax.ShapeDtypeStruct((n, ho), jnp.float32),
          jax.ShapeDtypeStruct((t, 2, ho), jnp.float32),
      ],
  )(x, w, b.reshape(1, ho))


def _combine(h, wr_eff, bl_eff, wl_st, aggs, n):
  nrel = wl_st.shape[0]
  t = n // _BR

  def body(*refs):
    h_ref, wr_ref, bl_ref, wl_ref = refs[0:4]
    agg_refs = refs[4:4 + nrel]
    z_ref, st_ref = refs[4 + nrel:]
    z = jnp.dot(h_ref[...], wr_ref[...].T,
                preferred_element_type=jnp.float32) + bl_ref[...]
    for r in range(nrel):
      a = agg_refs[r][...]
      cnt = jnp.maximum(a[:, 64:65], 1.0)
      mean = a / cnt
      z = z + jnp.dot(mean, wl_ref[r].T, preferred_element_type=jnp.float32)
    z_ref[...] = z
    st_ref[...] = jnp.stack([jnp.sum(z, 0), jnp.sum(z * z, 0)])[None]

  in_specs = [
      pl.BlockSpec((_BR, _W), lambda i: (i, 0)),
      pl.BlockSpec((_W, _W), lambda i: (0, 0)),
      pl.BlockSpec((1, _W), lambda i: (0, 0)),
      pl.BlockSpec((nrel, _W, _W), lambda i: (0, 0, 0)),
  ]
  in_specs += [pl.BlockSpec((_BR, _W), lambda i: (i, 0))] * nrel

  return pl.pallas_call(
      body,
      grid=(t,),
      in_specs=in_specs,
      out_specs=[
          pl.BlockSpec((_BR, _W), lambda i: (i, 0)),
          pl.BlockSpec((1, 2, _W), lambda i: (i, 0, 0)),
      ],
      out_shape=[
          jax.ShapeDtypeStruct((n, _W), jnp.float32),
          jax.ShapeDtypeStruct((t, 2, _W), jnp.float32),
      ],
  )(h, wr_eff, bl_eff.reshape(1, _W), wl_st, *aggs)


def _apply_bn_relu(z, st, g, beta, n):
  ho = z.shape[1]
  t = n // _BR

  def body(z_ref, st_ref, g_ref, b_ref, h_ref):
    stf = st_ref[...]
    inv_n = 1.0 / n
    m = jnp.sum(stf[:, 0, :], 0) * inv_n
    ms2 = jnp.sum(stf[:, 1, :], 0) * inv_n
    var = ms2 - m * m
    sc = g_ref[...][0] * lax.rsqrt(var + 1e-5)
    sh = b_ref[...][0] - m * sc
    h_ref[...] = jnp.maximum(z_ref[...] * sc + sh, 0.0)

  return pl.pallas_call(
      body,
      grid=(t,),
      in_specs=[
          pl.BlockSpec((_BR, ho), lambda i: (i, 0)),
          pl.BlockSpec((t, 2, ho), lambda i: (0, 0, 0)),
          pl.BlockSpec((1, ho), lambda i: (0, 0)),
          pl.BlockSpec((1, ho), lambda i: (0, 0)),
      ],
      out_specs=pl.BlockSpec((_BR, ho), lambda i: (i, 0)),
      out_shape=jax.ShapeDtypeStruct((n, ho), jnp.float32),
  )(z, st, g.reshape(1, ho), beta.reshape(1, ho))


def _apply_bn_relu_sig(z, st, g, beta, w3p, b3p, n):
  ho = z.shape[1]
  t = n // _BR

  def body(z_ref, st_ref, g_ref, b_ref, w3_ref, b3_ref, o_ref):
    stf = st_ref[...]
    inv_n = 1.0 / n
    m = jnp.sum(stf[:, 0, :], 0) * inv_n
    ms2 = jnp.sum(stf[:, 1, :], 0) * inv_n
    var = ms2 - m * m
    sc = g_ref[...][0] * lax.rsqrt(var + 1e-5)
    sh = b_ref[...][0] - m * sc
    h2 = jnp.maximum(z_ref[...] * sc + sh, 0.0)
    o = jnp.dot(h2, w3_ref[...].T, preferred_element_type=jnp.float32)
    o_ref[...] = jax.nn.sigmoid(o + b3_ref[...])

  return pl.pallas_call(
      body,
      grid=(t,),
      in_specs=[
          pl.BlockSpec((_BR, ho), lambda i: (i, 0)),
          pl.BlockSpec((t, 2, ho), lambda i: (0, 0, 0)),
          pl.BlockSpec((1, ho), lambda i: (0, 0)),
          pl.BlockSpec((1, ho), lambda i: (0, 0)),
          pl.BlockSpec((8, ho), lambda i: (0, 0)),
          pl.BlockSpec((1, 8), lambda i: (0, 0)),
      ],
      out_specs=pl.BlockSpec((_BR, 8), lambda i: (i, 0)),
      out_shape=jax.ShapeDtypeStruct((n, 8), jnp.float32),
  )(z, st, g.reshape(1, ho), beta.reshape(1, ho), w3p, b3p)


@functools.cache
def _get_sc_bin():
  return _sc_bin_kernel()


@functools.cache
def _get_sc_agg():
  return _sc_agg_kernel()


def _pad_edges(ei, n_src, n_dst):
  padn = _EP - _E
  pad_src = (jnp.arange(padn, dtype=jnp.int32) * 8) % n_src
  pad_dst = jnp.full((padn,), n_dst, jnp.int32)
  return (jnp.concatenate([ei[0], pad_src]),
          jnp.concatenate([ei[1], pad_dst]))


def _pad_cols(a, w):
  pad = [(0, 0)] * (a.ndim - 1) + [(0, w - a.shape[-1])]
  return jnp.pad(a, pad)


def kernel(x_customer, x_transaction, x_device, x_email, x_address,
           ei_makes, ei_used_in, ei_linked_to, ei_located_at,
           ei_rev_makes, ei_rev_used_in, ei_rev_linked_to,
           ei_rev_located_at, params):
  xs = {"customer": x_customer, "transaction": x_transaction,
        "device": x_device, "email": x_email, "address": x_address}
  eis = {"makes": ei_makes, "used_in": ei_used_in,
         "linked_to": ei_linked_to, "located_at": ei_located_at,
         "rev_makes": ei_rev_makes, "rev_used_in": ei_rev_used_in,
         "rev_linked_to": ei_rev_linked_to,
         "rev_located_at": ei_rev_located_at}

  marker = jnp.zeros((_W,), jnp.float32).at[64].set(1.0)

  def bn128(g, beta):
    return _pad_cols(g, _W), _pad_cols(beta, _W) + marker

  h = {}
  for nt in _NT:
    pp = params["in_proj"][nt]
    z, st = _proj(xs[nt], _pad_cols(pp["W"].T, _W).T, _pad_cols(pp["b"], _W))
    g128, beta128 = bn128(pp["g"], pp["beta"])
    h[nt] = _apply_bn_relu(z, st, g128, beta128, _NN[nt])

  eip = [_pad_edges(eis[r], _NN[s], _NN[d]) for (r, s, d) in _RELS]
  packed = _get_sc_bin()(*[e[0] for e in eip], *[e[1] for e in eip])
  agg = _get_sc_agg()

  for layer in ["1", "2"]:
    outs = agg(*[h[nt] for nt in _NT], *packed)
    cv = params["conv" + layer]
    bn = params["bn" + layer]
    newh = {}
    for nt in _NT:
      ridx = [i for i, (_, _, d) in enumerate(_RELS) if d == nt]
      rnames = [_RELS[i][0] for i in ridx]

      def wpad(w):
        return jnp.pad(w, ((0, _W - _H), (0, _W - _H)))

      wl_st = jnp.stack([wpad(cv[r]["Wl"]) for r in rnames])
      wr_eff = wpad(sum(cv[r]["Wr"] for r in rnames))
      bl_eff = _pad_cols(sum(cv[r]["bl"] for r in rnames), _W)
      aggs = [outs[i] for i in ridx]
      z, st = _combine(h[nt], wr_eff, bl_eff, wl_st, aggs, _NN[nt])
      g128, beta128 = bn128(bn[nt]["g"], bn[nt]["b"])
      newh[nt] = _apply_bn_relu(z, st, g128, beta128, _NN[nt])
    h = newh

  c = params["cls"]
  nt_n = _NN["transaction"]
  z1, st1 = _proj(h["transaction"], _pad_cols(c["W1"], _W), c["b1"])
  h1 = _apply_bn_relu(z1, st1, c["g1"], c["beta1"], nt_n)
  z2, st2 = _proj(h1, c["W2"], c["b2"])
  w3p = jnp.zeros((8, c["W3"].shape[1]), jnp.float32).at[0].set(c["W3"][0])
  b3p = jnp.zeros((1, 8), jnp.float32).at[0, 0].set(c["b3"][0])
  out8 = _apply_bn_relu_sig(z2, st2, c["g2"], c["beta2"], w3p, b3p, nt_n)
  return out8[:, 0]


# speedup vs baseline: 2.6891x; 1.0417x over previous
import functools

import jax
import jax.numpy as jnp
from jax import lax
from jax.experimental import pallas as pl
from jax.experimental.pallas import tpu as pltpu
from jax.experimental.pallas import tpu_sc as plsc

_NT = ["customer", "transaction", "device", "email", "address"]
_NN = {"customer": 50000, "transaction": 100000, "device": 10000,
       "email": 10000, "address": 10000}
_H = 64
_W = 128
_E = 150000
_RELS = [
    ("makes", "customer", "transaction"),
    ("used_in", "device", "transaction"),
    ("linked_to", "email", "transaction"),
    ("located_at", "address", "transaction"),
    ("rev_makes", "transaction", "customer"),
    ("rev_used_in", "transaction", "device"),
    ("rev_linked_to", "transaction", "email"),
    ("rev_located_at", "transaction", "address"),
]

_NSUB = 16
_EPT = 9472
_EP = _NSUB * _EPT
_NBLK = _EPT // 128
_CHUNK = {"transaction": (8448, 6, 101376),
          "customer": (8448, 3, 50688),
          "device": (5120, 1, 10240),
          "email": (5120, 1, 10240),
          "address": (5120, 1, 10240)}
_ACC_ROWS = 8464
_ZROWS = 64


_EW = 2368
_NWIN = _EPT // _EW
_PCAP = _EPT + 256


def _sc_bin_kernel():
  mesh = plsc.VectorSubcoreMesh(core_axis_name="c", subcore_axis_name="s")
  out_type = []
  for (_, _, d) in _RELS:
    m = _CHUNK[d][1]
    out_type += [
        jax.ShapeDtypeStruct((2 * m * _NSUB * _PCAP,), jnp.int32),
        jax.ShapeDtypeStruct((2 * m * _NSUB * _PCAP,), jnp.int32),
        jax.ShapeDtypeStruct((2 * m * _NSUB * 16,), jnp.int32),
    ]

  scratch = [
      pltpu.VMEM((_EW,), jnp.int32),
      pltpu.VMEM((_EW,), jnp.int32),
      pltpu.VMEM((_PCAP,), jnp.int32),
      pltpu.VMEM((_PCAP,), jnp.int32),
      pltpu.VMEM((16,), jnp.int32),
  ]

  def body(*refs):
    src_refs = refs[0:8]
    dst_refs = refs[8:16]
    out_refs = refs[16:40]
    (srcw, dstw, psrc, pldst, cntv) = refs[40:]

    s_idx = lax.axis_index("s")
    c_idx = lax.axis_index("c")
    zi = jnp.zeros((16,), jnp.int32)
    lane = lax.iota(jnp.int32, 16)

    for ri, (_, _, d_nt) in enumerate(_RELS):
      cr, m, _ = _CHUNK[d_nt]
      ps_ref, pd_ref, cn_ref = out_refs[3 * ri:3 * ri + 3]
      stripe0 = s_idx * _EPT

      def chunk(k, _):
        g = 2 * k + c_idx
        lo = g * cr

        def cvec_outer(i, off):
          def cvec(i, off):
            d = dstw[pl.ds(i * 16, 16)]
            s = srcw[pl.ds(i * 16, 16)]
            inr = (d >= lo) & (d < lo + cr)
            inc = jnp.where(inr, jnp.int32(1), jnp.int32(0))
            pos = jnp.where(inr, plsc.cumsum(inc) - 1 + off, _EPT + 240)
            plsc.store_scatter(psrc, [pos], s)
            plsc.store_scatter(pldst, [pos], d - lo)
            return off + jnp.sum(inc)
          return lax.fori_loop(0, _EW // 16, cvec, off)

        def win(w, off):
          pltpu.sync_copy(src_refs[ri].at[pl.ds(stripe0 + w * _EW, _EW)],
                          srcw)
          pltpu.sync_copy(dst_refs[ri].at[pl.ds(stripe0 + w * _EW, _EW)],
                          dstw)
          return cvec_outer(w, off)
        off = lax.fori_loop(0, _NWIN, win, jnp.int32(0))

        for j in range(8):
          pos = off + j * 16 + lane
          plsc.store_scatter(psrc, [pos], zi)
          plsc.store_scatter(pldst, [pos], jnp.full((16,), cr, jnp.int32))

        base = (g * _NSUB + s_idx) * _PCAP
        pltpu.sync_copy(psrc, ps_ref.at[pl.ds(base, _PCAP)])
        pltpu.sync_copy(pldst, pd_ref.at[pl.ds(base, _PCAP)])
        cntv[pl.ds(0, 16)] = jnp.where(lane == 0, zi + off, zi)
        pltpu.sync_copy(cntv, cn_ref.at[pl.ds((g * _NSUB + s_idx) * 16, 16)])
        return 0

      lax.fori_loop(0, m, chunk, 0)

  return pl.kernel(
      body, out_type=out_type, mesh=mesh, scratch_types=scratch,
      compiler_params=pltpu.CompilerParams(needs_layout_passes=False))


def _sc_agg_kernel():
  mesh = plsc.VectorSubcoreMesh(core_axis_name="c", subcore_axis_name="s")
  out_type = [jax.ShapeDtypeStruct((_CHUNK[d][2], _W), jnp.float32)
              for (_, _, d) in _RELS]

  scratch = [
      pltpu.VMEM_SHARED((_ACC_ROWS, _W), jnp.float32),
      pltpu.VMEM((_PCAP,), jnp.int32),
      pltpu.VMEM((_PCAP,), jnp.int32),
      pltpu.VMEM((16,), jnp.int32),
      pltpu.VMEM((2, 128), jnp.int32),
      pltpu.VMEM((128, _W), jnp.float32),
      pltpu.VMEM((128, _W), jnp.float32),
      pltpu.VMEM((_ZROWS, _W), jnp.float32),
      pltpu.SemaphoreType.DMA,
      pltpu.SemaphoreType.DMA,
      pltpu.SemaphoreType.DMA,
  ]

  def body(*refs):
    h_refs = refs[0:5]
    pk_refs = refs[5:29]
    out_refs = refs[29:37]
    (acc_sh, psrc, pldst, cntv, pdst, rows0, rows1, zb,
     sem0, sem1, sem2) = refs[37:]

    s_idx = lax.axis_index("s")
    c_idx = lax.axis_index("c")

    zf = jnp.zeros((16,), jnp.float32)

    def fill_zb(r, _):
      for l in range(_W // 16):
        zb[r, pl.ds(l * 16, 16)] = zf
      return 0
    lax.fori_loop(0, _ZROWS, fill_zb, 0)

    h_by_nt = dict(zip(_NT, h_refs))

    for ri, (_, s_nt, d_nt) in enumerate(_RELS):
      cr, m, _ = _CHUNK[d_nt]
      h_ref = h_by_nt[s_nt]
      ps_ref, pd_ref, cn_ref = pk_refs[3 * ri:3 * ri + 3]
      agg_ref = out_refs[ri]
      rpt = cr // _NSUB
      share0 = s_idx * rpt

      def pk_copies(k):
        g = 2 * k + c_idx
        base = (g * _NSUB + s_idx) * _PCAP
        return [
            pltpu.make_async_copy(ps_ref.at[pl.ds(base, _PCAP)], psrc, sem2),
            pltpu.make_async_copy(pd_ref.at[pl.ds(base, _PCAP)], pldst, sem2),
            pltpu.make_async_copy(
                cn_ref.at[pl.ds((g * _NSUB + s_idx) * 16, 16)], cntv, sem2),
        ]

      for cp in pk_copies(0):
        cp.start()

      def chunk(k, _):
        g = 2 * k + c_idx
        lo = g * cr

        nfull, rem = divmod(rpt, _ZROWS)
        for z in range(nfull):
          pltpu.sync_copy(zb, acc_sh.at[pl.ds(share0 + z * _ZROWS, _ZROWS)])
        if rem:
          pltpu.sync_copy(zb.at[pl.ds(0, rem)],
                          acc_sh.at[pl.ds(share0 + nfull * _ZROWS, rem)])

        for cp in pk_copies(k):
          cp.wait()
        off = jnp.sum(cntv[pl.ds(0, 16)])
        nblk = (off + 127) // 128

        plsc.subcore_barrier()

        def start(b, rows, sem):
          pltpu.async_copy(h_ref.at[psrc.at[pl.ds(b * 128, 128)]], rows, sem)

        def drain(b, j, rows, sem):
          for l in range(8):
            pdst[j, pl.ds(l * 16, 16)] = pldst[pl.ds(b * 128 + l * 16, 16)]
          pltpu.make_async_copy(h_ref.at[psrc.at[pl.ds(b * 128, 128)]],
                                rows, sem).wait()
          pltpu.sync_copy(rows, acc_sh.at[pdst.at[j]], add=True)

        @pl.when(nblk > 0)
        def _p0():
          start(0, rows0, sem0)

        def pair(g, _):
          b0 = 2 * g
          b1 = b0 + 1

          @pl.when(b1 < nblk)
          def _s1():
            start(b1, rows1, sem1)

          @pl.when(b0 < nblk)
          def _d0():
            drain(b0, 0, rows0, sem0)

          @pl.when(b1 + 1 < nblk)
          def _s2():
            start(b1 + 1, rows0, sem0)

          @pl.when(b1 < nblk)
          def _d1():
            drain(b1, 1, rows1, sem1)
          return 0
        lax.fori_loop(0, (_NBLK + 1) // 2, pair, 0)

        @pl.when(k + 1 < m)
        def _pf():
          for cp in pk_copies(k + 1):
            cp.start()
        plsc.subcore_barrier()

        pltpu.sync_copy(acc_sh.at[pl.ds(share0, rpt)],
                        agg_ref.at[pl.ds(lo + share0, rpt)])
        return 0

      lax.fori_loop(0, m, chunk, 0)

  return pl.kernel(
      body, out_type=out_type, mesh=mesh, scratch_types=scratch,
      compiler_params=pltpu.CompilerParams(needs_layout_passes=False))


_BR = 1000


def _proj(x, w, b):
  n, kd = x.shape
  ho = w.shape[0]
  t = n // _BR

  def body(x_ref, w_ref, b_ref, z_ref, st_ref):
    z = jnp.dot(x_ref[...], w_ref[...].T,
                preferred_element_type=jnp.float32) + b_ref[...]
    z_ref[...] = z
    st_ref[...] = jnp.stack([jnp.sum(z, 0), jnp.sum(z * z, 0)])[None]

  return pl.pallas_call(
      body,
      grid=(t,),
      in_specs=[
          pl.BlockSpec((_BR, kd), lambda i: (i, 0)),
          pl.BlockSpec((ho, kd), lambda i: (0, 0)),
          pl.BlockSpec((1, ho), lambda i: (0, 0)),
      ],
      out_specs=[
          pl.BlockSpec((_BR, ho), lambda i: (i, 0)),
          pl.BlockSpec((1, 2, ho), lambda i: (i, 0, 0)),
      ],
      out_shape=[
          jax.ShapeDtypeStruct((n, ho), jnp.float32),
          jax.ShapeDtypeStruct((t, 2, ho), jnp.float32),
      ],
  )(x, w, b.reshape(1, ho))


def _combine(h, wr_eff, bl_eff, wl_st, aggs, n):
  nrel = wl_st.shape[0]
  t = n // _BR

  def body(*refs):
    h_ref, wr_ref, bl_ref, wl_ref = refs[0:4]
    agg_refs = refs[4:4 + nrel]
    z_ref, st_ref = refs[4 + nrel:]
    z = jnp.dot(h_ref[...], wr_ref[...].T,
                preferred_element_type=jnp.float32) + bl_ref[...]
    for r in range(nrel):
      a = agg_refs[r][...]
      cnt = jnp.maximum(a[:, 64:65], 1.0)
      mean = a / cnt
      z = z + jnp.dot(mean, wl_ref[r].T, preferred_element_type=jnp.float32)
    z_ref[...] = z
    st_ref[...] = jnp.stack([jnp.sum(z, 0), jnp.sum(z * z, 0)])[None]

  in_specs = [
      pl.BlockSpec((_BR, _W), lambda i: (i, 0)),
      pl.BlockSpec((_W, _W), lambda i: (0, 0)),
      pl.BlockSpec((1, _W), lambda i: (0, 0)),
      pl.BlockSpec((nrel, _W, _W), lambda i: (0, 0, 0)),
  ]
  in_specs += [pl.BlockSpec((_BR, _W), lambda i: (i, 0))] * nrel

  return pl.pallas_call(
      body,
      grid=(t,),
      in_specs=in_specs,
      out_specs=[
          pl.BlockSpec((_BR, _W), lambda i: (i, 0)),
          pl.BlockSpec((1, 2, _W), lambda i: (i, 0, 0)),
      ],
      out_shape=[
          jax.ShapeDtypeStruct((n, _W), jnp.float32),
          jax.ShapeDtypeStruct((t, 2, _W), jnp.float32),
      ],
  )(h, wr_eff, bl_eff.reshape(1, _W), wl_st, *aggs)


def _apply_bn_relu(z, st, g, beta, n):
  ho = z.shape[1]
  t = n // _BR

  def body(z_ref, st_ref, g_ref, b_ref, h_ref):
    stf = st_ref[...]
    inv_n = 1.0 / n
    m = jnp.sum(stf[:, 0, :], 0) * inv_n
    ms2 = jnp.sum(stf[:, 1, :], 0) * inv_n
    var = ms2 - m * m
    sc = g_ref[...][0] * lax.rsqrt(var + 1e-5)
    sh = b_ref[...][0] - m * sc
    h_ref[...] = jnp.maximum(z_ref[...] * sc + sh, 0.0)

  return pl.pallas_call(
      body,
      grid=(t,),
      in_specs=[
          pl.BlockSpec((_BR, ho), lambda i: (i, 0)),
          pl.BlockSpec((t, 2, ho), lambda i: (0, 0, 0)),
          pl.BlockSpec((1, ho), lambda i: (0, 0)),
          pl.BlockSpec((1, ho), lambda i: (0, 0)),
      ],
      out_specs=pl.BlockSpec((_BR, ho), lambda i: (i, 0)),
      out_shape=jax.ShapeDtypeStruct((n, ho), jnp.float32),
  )(z, st, g.reshape(1, ho), beta.reshape(1, ho))


def _apply_bn_relu_sig(z, st, g, beta, w3p, b3p, n):
  ho = z.shape[1]
  t = n // _BR

  def body(z_ref, st_ref, g_ref, b_ref, w3_ref, b3_ref, o_ref):
    stf = st_ref[...]
    inv_n = 1.0 / n
    m = jnp.sum(stf[:, 0, :], 0) * inv_n
    ms2 = jnp.sum(stf[:, 1, :], 0) * inv_n
    var = ms2 - m * m
    sc = g_ref[...][0] * lax.rsqrt(var + 1e-5)
    sh = b_ref[...][0] - m * sc
    h2 = jnp.maximum(z_ref[...] * sc + sh, 0.0)
    o = jnp.dot(h2, w3_ref[...].T, preferred_element_type=jnp.float32)
    o_ref[...] = jax.nn.sigmoid(o + b3_ref[...])

  return pl.pallas_call(
      body,
      grid=(t,),
      in_specs=[
          pl.BlockSpec((_BR, ho), lambda i: (i, 0)),
          pl.BlockSpec((t, 2, ho), lambda i: (0, 0, 0)),
          pl.BlockSpec((1, ho), lambda i: (0, 0)),
          pl.BlockSpec((1, ho), lambda i: (0, 0)),
          pl.BlockSpec((8, ho), lambda i: (0, 0)),
          pl.BlockSpec((1, 8), lambda i: (0, 0)),
      ],
      out_specs=pl.BlockSpec((_BR, 8), lambda i: (i, 0)),
      out_shape=jax.ShapeDtypeStruct((n, 8), jnp.float32),
  )(z, st, g.reshape(1, ho), beta.reshape(1, ho), w3p, b3p)


@functools.cache
def _get_sc_bin():
  return _sc_bin_kernel()


@functools.cache
def _get_sc_agg():
  return _sc_agg_kernel()


def _pad_edges(ei, n_src, n_dst):
  padn = _EP - _E
  pad_src = (jnp.arange(padn, dtype=jnp.int32) * 8) % n_src
  pad_dst = jnp.full((padn,), n_dst, jnp.int32)
  return (jnp.concatenate([ei[0], pad_src]),
          jnp.concatenate([ei[1], pad_dst]))


def _pad_cols(a, w):
  pad = [(0, 0)] * (a.ndim - 1) + [(0, w - a.shape[-1])]
  return jnp.pad(a, pad)


def kernel(x_customer, x_transaction, x_device, x_email, x_address,
           ei_makes, ei_used_in, ei_linked_to, ei_located_at,
           ei_rev_makes, ei_rev_used_in, ei_rev_linked_to,
           ei_rev_located_at, params):
  xs = {"customer": x_customer, "transaction": x_transaction,
        "device": x_device, "email": x_email, "address": x_address}
  eis = {"makes": ei_makes, "used_in": ei_used_in,
         "linked_to": ei_linked_to, "located_at": ei_located_at,
         "rev_makes": ei_rev_makes, "rev_used_in": ei_rev_used_in,
         "rev_linked_to": ei_rev_linked_to,
         "rev_located_at": ei_rev_located_at}

  marker = jnp.zeros((_W,), jnp.float32).at[64].set(1.0)

  def bn128(g, beta):
    return _pad_cols(g, _W), _pad_cols(beta, _W) + marker

  h = {}
  for nt in _NT:
    pp = params["in_proj"][nt]
    z, st = _proj(xs[nt], _pad_cols(pp["W"].T, _W).T, _pad_cols(pp["b"], _W))
    g128, beta128 = bn128(pp["g"], pp["beta"])
    h[nt] = _apply_bn_relu(z, st, g128, beta128, _NN[nt])

  eip = [_pad_edges(eis[r], _NN[s], _NN[d]) for (r, s, d) in _RELS]
  packed = _get_sc_bin()(*[e[0] for e in eip], *[e[1] for e in eip])
  agg = _get_sc_agg()

  for layer in ["1", "2"]:
    outs = agg(*[h[nt] for nt in _NT], *packed)
    cv = params["conv" + layer]
    bn = params["bn" + layer]
    newh = {}
    for nt in _NT:
      ridx = [i for i, (_, _, d) in enumerate(_RELS) if d == nt]
      rnames = [_RELS[i][0] for i in ridx]

      def wpad(w):
        return jnp.pad(w, ((0, _W - _H), (0, _W - _H)))

      wl_st = jnp.stack([wpad(cv[r]["Wl"]) for r in rnames])
      wr_eff = wpad(sum(cv[r]["Wr"] for r in rnames))
      bl_eff = _pad_cols(sum(cv[r]["bl"] for r in rnames), _W)
      aggs = [outs[i] for i in ridx]
      z, st = _combine(h[nt], wr_eff, bl_eff, wl_st, aggs, _NN[nt])
      g128, beta128 = bn128(bn[nt]["g"], bn[nt]["b"])
      newh[nt] = _apply_bn_relu(z, st, g128, beta128, _NN[nt])
    h = newh

  c = params["cls"]
  nt_n = _NN["transaction"]
  z1, st1 = _proj(h["transaction"], _pad_cols(c["W1"], _W), c["b1"])
  h1 = _apply_bn_relu(z1, st1, c["g1"], c["beta1"], nt_n)
  z2, st2 = _proj(h1, c["W2"], c["b2"])
  w3p = jnp.zeros((8, c["W3"].shape[1]), jnp.float32).at[0].set(c["W3"][0])
  b3p = jnp.zeros((1, 8), jnp.float32).at[0, 0].set(c["b3"][0])
  out8 = _apply_bn_relu_sig(z2, st2, c["g2"], c["beta2"], w3p, b3p, nt_n)
  return out8[:, 0]


# speedup vs baseline: 2.6969x; 1.0029x over previous
import functools

import jax
import jax.numpy as jnp
from jax import lax
from jax.experimental import pallas as pl
from jax.experimental.pallas import tpu as pltpu
from jax.experimental.pallas import tpu_sc as plsc

_NT = ["customer", "transaction", "device", "email", "address"]
_NN = {"customer": 50000, "transaction": 100000, "device": 10000,
       "email": 10000, "address": 10000}
_H = 64
_W = 128
_E = 150000
_RELS = [
    ("makes", "customer", "transaction"),
    ("used_in", "device", "transaction"),
    ("linked_to", "email", "transaction"),
    ("located_at", "address", "transaction"),
    ("rev_makes", "transaction", "customer"),
    ("rev_used_in", "transaction", "device"),
    ("rev_linked_to", "transaction", "email"),
    ("rev_located_at", "transaction", "address"),
]

_NSUB = 16
_EPT = 9472
_EP = _NSUB * _EPT
_NBLK = _EPT // 128
_CHUNK = {"transaction": (8448, 6, 101376),
          "customer": (8448, 3, 50688),
          "device": (5120, 1, 10240),
          "email": (5120, 1, 10240),
          "address": (5120, 1, 10240)}
_ACC_ROWS = 8464
_ZROWS = 64


_EW = 2368
_NWIN = _EPT // _EW
_PCAP = _EPT + 256


def _sc_bin_kernel():
  mesh = plsc.VectorSubcoreMesh(core_axis_name="c", subcore_axis_name="s")
  out_type = []
  for (_, _, d) in _RELS:
    m = _CHUNK[d][1]
    out_type += [
        jax.ShapeDtypeStruct((2 * m * _NSUB * _PCAP,), jnp.int32),
        jax.ShapeDtypeStruct((2 * m * _NSUB * _PCAP,), jnp.int32),
        jax.ShapeDtypeStruct((2 * m * _NSUB * 16,), jnp.int32),
    ]

  scratch = [
      pltpu.VMEM((_EW,), jnp.int32),
      pltpu.VMEM((_EW,), jnp.int32),
      pltpu.VMEM((_PCAP,), jnp.int32),
      pltpu.VMEM((_PCAP,), jnp.int32),
      pltpu.VMEM((16,), jnp.int32),
  ]

  def body(*refs):
    src_refs = refs[0:8]
    dst_refs = refs[8:16]
    out_refs = refs[16:40]
    (srcw, dstw, psrc, pldst, cntv) = refs[40:]

    s_idx = lax.axis_index("s")
    c_idx = lax.axis_index("c")
    zi = jnp.zeros((16,), jnp.int32)
    lane = lax.iota(jnp.int32, 16)

    for ri, (_, _, d_nt) in enumerate(_RELS):
      cr, m, _ = _CHUNK[d_nt]
      ps_ref, pd_ref, cn_ref = out_refs[3 * ri:3 * ri + 3]
      stripe0 = s_idx * _EPT

      def chunk(k, _):
        g = 2 * k + c_idx
        lo = g * cr

        def cvec_outer(i, off):
          def cvec(i, off):
            d = dstw[pl.ds(i * 16, 16)]
            s = srcw[pl.ds(i * 16, 16)]
            inr = (d >= lo) & (d < lo + cr)
            inc = jnp.where(inr, jnp.int32(1), jnp.int32(0))
            pos = jnp.where(inr, plsc.cumsum(inc) - 1 + off, _EPT + 240)
            plsc.store_scatter(psrc, [pos], s)
            plsc.store_scatter(pldst, [pos], d - lo)
            return off + jnp.sum(inc)
          return lax.fori_loop(0, _EW // 16, cvec, off)

        def win(w, off):
          pltpu.sync_copy(src_refs[ri].at[pl.ds(stripe0 + w * _EW, _EW)],
                          srcw)
          pltpu.sync_copy(dst_refs[ri].at[pl.ds(stripe0 + w * _EW, _EW)],
                          dstw)
          return cvec_outer(w, off)
        off = lax.fori_loop(0, _NWIN, win, jnp.int32(0))

        for j in range(8):
          pos = off + j * 16 + lane
          plsc.store_scatter(psrc, [pos], zi)
          plsc.store_scatter(pldst, [pos], jnp.full((16,), cr, jnp.int32))

        base = (g * _NSUB + s_idx) * _PCAP
        pltpu.sync_copy(psrc, ps_ref.at[pl.ds(base, _PCAP)])
        pltpu.sync_copy(pldst, pd_ref.at[pl.ds(base, _PCAP)])
        cntv[pl.ds(0, 16)] = jnp.where(lane == 0, zi + off, zi)
        pltpu.sync_copy(cntv, cn_ref.at[pl.ds((g * _NSUB + s_idx) * 16, 16)])
        return 0

      lax.fori_loop(0, m, chunk, 0)

  return pl.kernel(
      body, out_type=out_type, mesh=mesh, scratch_types=scratch,
      compiler_params=pltpu.CompilerParams(needs_layout_passes=False))


def _sc_agg_kernel():
  mesh = plsc.VectorSubcoreMesh(core_axis_name="c", subcore_axis_name="s")
  out_type = [jax.ShapeDtypeStruct((_CHUNK[d][2], _W), jnp.float32)
              for (_, _, d) in _RELS]

  scratch = [
      pltpu.VMEM_SHARED((_ACC_ROWS, _W), jnp.float32),
      pltpu.VMEM((_PCAP,), jnp.int32),
      pltpu.VMEM((_PCAP,), jnp.int32),
      pltpu.VMEM((16,), jnp.int32),
      pltpu.VMEM((2, 128), jnp.int32),
      pltpu.VMEM((128, _W), jnp.float32),
      pltpu.VMEM((128, _W), jnp.float32),
      pltpu.VMEM((_ZROWS, _W), jnp.float32),
      pltpu.SemaphoreType.DMA,
      pltpu.SemaphoreType.DMA,
      pltpu.SemaphoreType.DMA,
  ]

  def body(*refs):
    h_refs = refs[0:5]
    pk_refs = refs[5:29]
    out_refs = refs[29:37]
    (acc_sh, psrc, pldst, cntv, pdst, rows0, rows1, zb,
     sem0, sem1, sem2) = refs[37:]

    s_idx = lax.axis_index("s")
    c_idx = lax.axis_index("c")

    zf = jnp.zeros((16,), jnp.float32)

    def fill_zb(r, _):
      for l in range(_W // 16):
        zb[r, pl.ds(l * 16, 16)] = zf
      return 0
    lax.fori_loop(0, _ZROWS, fill_zb, 0)

    h_by_nt = dict(zip(_NT, h_refs))

    for ri, (_, s_nt, d_nt) in enumerate(_RELS):
      cr, m, _ = _CHUNK[d_nt]
      h_ref = h_by_nt[s_nt]
      ps_ref, pd_ref, cn_ref = pk_refs[3 * ri:3 * ri + 3]
      agg_ref = out_refs[ri]
      rpt = cr // _NSUB
      share0 = s_idx * rpt

      def pk_copies(k):
        g = 2 * k + c_idx
        base = (g * _NSUB + s_idx) * _PCAP
        return [
            pltpu.make_async_copy(ps_ref.at[pl.ds(base, _PCAP)], psrc, sem2),
            pltpu.make_async_copy(pd_ref.at[pl.ds(base, _PCAP)], pldst, sem2),
            pltpu.make_async_copy(
                cn_ref.at[pl.ds((g * _NSUB + s_idx) * 16, 16)], cntv, sem2),
        ]

      for cp in pk_copies(0):
        cp.start()

      def chunk(k, _):
        g = 2 * k + c_idx
        lo = g * cr

        nfull, rem = divmod(rpt, _ZROWS)
        zcps = [pltpu.make_async_copy(
            zb, acc_sh.at[pl.ds(share0 + z * _ZROWS, _ZROWS)], sem0)
            for z in range(nfull)]
        if rem:
          zcps.append(pltpu.make_async_copy(
              zb.at[pl.ds(0, rem)],
              acc_sh.at[pl.ds(share0 + nfull * _ZROWS, rem)], sem0))
        for cp in zcps:
          cp.start()

        for cp in pk_copies(k):
          cp.wait()
        off = jnp.sum(cntv[pl.ds(0, 16)])
        nblk = (off + 127) // 128

        for cp in zcps:
          cp.wait()

        plsc.subcore_barrier()

        def start(b, rows, sem):
          pltpu.async_copy(h_ref.at[psrc.at[pl.ds(b * 128, 128)]], rows, sem)

        def drain(b, j, rows, sem):
          for l in range(8):
            pdst[j, pl.ds(l * 16, 16)] = pldst[pl.ds(b * 128 + l * 16, 16)]
          pltpu.make_async_copy(h_ref.at[psrc.at[pl.ds(b * 128, 128)]],
                                rows, sem).wait()
          pltpu.sync_copy(rows, acc_sh.at[pdst.at[j]], add=True)

        @pl.when(nblk > 0)
        def _p0():
          start(0, rows0, sem0)

        def pair(g, _):
          b0 = 2 * g
          b1 = b0 + 1

          @pl.when(b1 < nblk)
          def _s1():
            start(b1, rows1, sem1)

          @pl.when(b0 < nblk)
          def _d0():
            drain(b0, 0, rows0, sem0)

          @pl.when(b1 + 1 < nblk)
          def _s2():
            start(b1 + 1, rows0, sem0)

          @pl.when(b1 < nblk)
          def _d1():
            drain(b1, 1, rows1, sem1)
          return 0
        lax.fori_loop(0, (_NBLK + 1) // 2, pair, 0)

        @pl.when(k + 1 < m)
        def _pf():
          for cp in pk_copies(k + 1):
            cp.start()
        plsc.subcore_barrier()

        pltpu.sync_copy(acc_sh.at[pl.ds(share0, rpt)],
                        agg_ref.at[pl.ds(lo + share0, rpt)])
        return 0

      lax.fori_loop(0, m, chunk, 0)

  return pl.kernel(
      body, out_type=out_type, mesh=mesh, scratch_types=scratch,
      compiler_params=pltpu.CompilerParams(needs_layout_passes=False))


_BR = 1000


def _proj(x, w, b):
  n, kd = x.shape
  ho = w.shape[0]
  t = n // _BR

  def body(x_ref, w_ref, b_ref, z_ref, st_ref):
    z = jnp.dot(x_ref[...], w_ref[...].T,
                preferred_element_type=jnp.float32) + b_ref[...]
    z_ref[...] = z
    st_ref[...] = jnp.stack([jnp.sum(z, 0), jnp.sum(z * z, 0)])[None]

  return pl.pallas_call(
      body,
      grid=(t,),
      in_specs=[
          pl.BlockSpec((_BR, kd), lambda i: (i, 0)),
          pl.BlockSpec((ho, kd), lambda i: (0, 0)),
          pl.BlockSpec((1, ho), lambda i: (0, 0)),
      ],
      out_specs=[
          pl.BlockSpec((_BR, ho), lambda i: (i, 0)),
          pl.BlockSpec((1, 2, ho), lambda i: (i, 0, 0)),
      ],
      out_shape=[
          jax.ShapeDtypeStruct((n, ho), jnp.float32),
          jax.ShapeDtypeStruct((t, 2, ho), jnp.float32),
      ],
  )(x, w, b.reshape(1, ho))


def _combine(h, wr_eff, bl_eff, wl_st, aggs, n):
  nrel = wl_st.shape[0]
  t = n // _BR

  def body(*refs):
    h_ref, wr_ref, bl_ref, wl_ref = refs[0:4]
    agg_refs = refs[4:4 + nrel]
    z_ref, st_ref = refs[4 + nrel:]
    z = jnp.dot(h_ref[...], wr_ref[...].T,
                preferred_element_type=jnp.float32) + bl_ref[...]
    for r in range(nrel):
      a = agg_refs[r][...]
      cnt = jnp.maximum(a[:, 64:65], 1.0)
      mean = a / cnt
      z = z + jnp.dot(mean, wl_ref[r].T, preferred_element_type=jnp.float32)
    z_ref[...] = z
    st_ref[...] = jnp.stack([jnp.sum(z, 0), jnp.sum(z * z, 0)])[None]

  in_specs = [
      pl.BlockSpec((_BR, _W), lambda i: (i, 0)),
      pl.BlockSpec((_W, _W), lambda i: (0, 0)),
      pl.BlockSpec((1, _W), lambda i: (0, 0)),
      pl.BlockSpec((nrel, _W, _W), lambda i: (0, 0, 0)),
  ]
  in_specs += [pl.BlockSpec((_BR, _W), lambda i: (i, 0))] * nrel

  return pl.pallas_call(
      body,
      grid=(t,),
      in_specs=in_specs,
      out_specs=[
          pl.BlockSpec((_BR, _W), lambda i: (i, 0)),
          pl.BlockSpec((1, 2, _W), lambda i: (i, 0, 0)),
      ],
      out_shape=[
          jax.ShapeDtypeStruct((n, _W), jnp.float32),
          jax.ShapeDtypeStruct((t, 2, _W), jnp.float32),
      ],
  )(h, wr_eff, bl_eff.reshape(1, _W), wl_st, *aggs)


def _apply_bn_relu(z, st, g, beta, n):
  ho = z.shape[1]
  t = n // _BR

  def body(z_ref, st_ref, g_ref, b_ref, h_ref):
    stf = st_ref[...]
    inv_n = 1.0 / n
    m = jnp.sum(stf[:, 0, :], 0) * inv_n
    ms2 = jnp.sum(stf[:, 1, :], 0) * inv_n
    var = ms2 - m * m
    sc = g_ref[...][0] * lax.rsqrt(var + 1e-5)
    sh = b_ref[...][0] - m * sc
    h_ref[...] = jnp.maximum(z_ref[...] * sc + sh, 0.0)

  return pl.pallas_call(
      body,
      grid=(t,),
      in_specs=[
          pl.BlockSpec((_BR, ho), lambda i: (i, 0)),
          pl.BlockSpec((t, 2, ho), lambda i: (0, 0, 0)),
          pl.BlockSpec((1, ho), lambda i: (0, 0)),
          pl.BlockSpec((1, ho), lambda i: (0, 0)),
      ],
      out_specs=pl.BlockSpec((_BR, ho), lambda i: (i, 0)),
      out_shape=jax.ShapeDtypeStruct((n, ho), jnp.float32),
  )(z, st, g.reshape(1, ho), beta.reshape(1, ho))


def _apply_bn_relu_sig(z, st, g, beta, w3p, b3p, n):
  ho = z.shape[1]
  t = n // _BR

  def body(z_ref, st_ref, g_ref, b_ref, w3_ref, b3_ref, o_ref):
    stf = st_ref[...]
    inv_n = 1.0 / n
    m = jnp.sum(stf[:, 0, :], 0) * inv_n
    ms2 = jnp.sum(stf[:, 1, :], 0) * inv_n
    var = ms2 - m * m
    sc = g_ref[...][0] * lax.rsqrt(var + 1e-5)
    sh = b_ref[...][0] - m * sc
    h2 = jnp.maximum(z_ref[...] * sc + sh, 0.0)
    o = jnp.dot(h2, w3_ref[...].T, preferred_element_type=jnp.float32)
    o_ref[...] = jax.nn.sigmoid(o + b3_ref[...])

  return pl.pallas_call(
      body,
      grid=(t,),
      in_specs=[
          pl.BlockSpec((_BR, ho), lambda i: (i, 0)),
          pl.BlockSpec((t, 2, ho), lambda i: (0, 0, 0)),
          pl.BlockSpec((1, ho), lambda i: (0, 0)),
          pl.BlockSpec((1, ho), lambda i: (0, 0)),
          pl.BlockSpec((8, ho), lambda i: (0, 0)),
          pl.BlockSpec((1, 8), lambda i: (0, 0)),
      ],
      out_specs=pl.BlockSpec((_BR, 8), lambda i: (i, 0)),
      out_shape=jax.ShapeDtypeStruct((n, 8), jnp.float32),
  )(z, st, g.reshape(1, ho), beta.reshape(1, ho), w3p, b3p)


@functools.cache
def _get_sc_bin():
  return _sc_bin_kernel()


@functools.cache
def _get_sc_agg():
  return _sc_agg_kernel()


def _pad_edges(ei, n_src, n_dst):
  padn = _EP - _E
  pad_src = (jnp.arange(padn, dtype=jnp.int32) * 8) % n_src
  pad_dst = jnp.full((padn,), n_dst, jnp.int32)
  return (jnp.concatenate([ei[0], pad_src]),
          jnp.concatenate([ei[1], pad_dst]))


def _pad_cols(a, w):
  pad = [(0, 0)] * (a.ndim - 1) + [(0, w - a.shape[-1])]
  return jnp.pad(a, pad)


def kernel(x_customer, x_transaction, x_device, x_email, x_address,
           ei_makes, ei_used_in, ei_linked_to, ei_located_at,
           ei_rev_makes, ei_rev_used_in, ei_rev_linked_to,
           ei_rev_located_at, params):
  xs = {"customer": x_customer, "transaction": x_transaction,
        "device": x_device, "email": x_email, "address": x_address}
  eis = {"makes": ei_makes, "used_in": ei_used_in,
         "linked_to": ei_linked_to, "located_at": ei_located_at,
         "rev_makes": ei_rev_makes, "rev_used_in": ei_rev_used_in,
         "rev_linked_to": ei_rev_linked_to,
         "rev_located_at": ei_rev_located_at}

  marker = jnp.zeros((_W,), jnp.float32).at[64].set(1.0)

  def bn128(g, beta):
    return _pad_cols(g, _W), _pad_cols(beta, _W) + marker

  h = {}
  for nt in _NT:
    pp = params["in_proj"][nt]
    z, st = _proj(xs[nt], _pad_cols(pp["W"].T, _W).T, _pad_cols(pp["b"], _W))
    g128, beta128 = bn128(pp["g"], pp["beta"])
    h[nt] = _apply_bn_relu(z, st, g128, beta128, _NN[nt])

  eip = [_pad_edges(eis[r], _NN[s], _NN[d]) for (r, s, d) in _RELS]
  packed = _get_sc_bin()(*[e[0] for e in eip], *[e[1] for e in eip])
  agg = _get_sc_agg()

  for layer in ["1", "2"]:
    outs = agg(*[h[nt] for nt in _NT], *packed)
    cv = params["conv" + layer]
    bn = params["bn" + layer]
    newh = {}
    for nt in _NT:
      ridx = [i for i, (_, _, d) in enumerate(_RELS) if d == nt]
      rnames = [_RELS[i][0] for i in ridx]

      def wpad(w):
        return jnp.pad(w, ((0, _W - _H), (0, _W - _H)))

      wl_st = jnp.stack([wpad(cv[r]["Wl"]) for r in rnames])
      wr_eff = wpad(sum(cv[r]["Wr"] for r in rnames))
      bl_eff = _pad_cols(sum(cv[r]["bl"] for r in rnames), _W)
      aggs = [outs[i] for i in ridx]
      z, st = _combine(h[nt], wr_eff, bl_eff, wl_st, aggs, _NN[nt])
      g128, beta128 = bn128(bn[nt]["g"], bn[nt]["b"])
      newh[nt] = _apply_bn_relu(z, st, g128, beta128, _NN[nt])
    h = newh

  c = params["cls"]
  nt_n = _NN["transaction"]
  z1, st1 = _proj(h["transaction"], _pad_cols(c["W1"], _W), c["b1"])
  h1 = _apply_bn_relu(z1, st1, c["g1"], c["beta1"], nt_n)
  z2, st2 = _proj(h1, c["W2"], c["b2"])
  w3p = jnp.zeros((8, c["W3"].shape[1]), jnp.float32).at[0].set(c["W3"][0])
  b3p = jnp.zeros((1, 8), jnp.float32).at[0, 0].set(c["b3"][0])
  out8 = _apply_bn_relu_sig(z2, st2, c["g2"], c["beta2"], w3p, b3p, nt_n)
  return out8[:, 0]
